# baseline (speedup 1.0000x reference)
import functools

import jax
import jax.numpy as jnp
from jax import lax
from jax.experimental import pallas as pl
from jax.experimental.pallas import tpu as pltpu
from jax.experimental.pallas import tpu_sc as plsc

NC = 2
NS = 16
NW = NC * NS
L = 16
B = 16384
D = 64
BPW = B // NW
NCH = BPW // L

_mesh = plsc.VectorSubcoreMesh(core_axis_name="c", subcore_axis_name="s",
                               num_cores=NC, num_subcores=NS)


@functools.partial(
    pl.kernel,
    out_type=jax.ShapeDtypeStruct((B,), jnp.float32),
    mesh=_mesh,
    scratch_types=[
        pltpu.VMEM((NCH, L), jnp.int32),
        pltpu.VMEM((NCH, L), jnp.int32),
        pltpu.VMEM((NCH, L), jnp.int32),
        pltpu.VMEM((NCH, L), jnp.int32),
        pltpu.VMEM((2 * L, 2 * D), jnp.float32),
        pltpu.VMEM((2 * L, 2 * D), jnp.float32),
        pltpu.VMEM((BPW,), jnp.float32),
        pltpu.VMEM((BPW,), jnp.float32),
        pltpu.VMEM((BPW,), jnp.float32),
        pltpu.VMEM((L,), jnp.float32),
        pltpu.SemaphoreType.DMA,
        pltpu.SemaphoreType.DMA,
    ],
    compiler_params=pltpu.CompilerParams(needs_layout_passes=False),
)
def _mf_kernel(user, item, ufp, ifp, ubias, ibias, gbias, out,
               uidx, iidx, upair, ipair, urows, irows, ubv, ibv, outv, gbv,
               bsem, gsem):
    wid = lax.axis_index("s") * NC + lax.axis_index("c")
    base = wid * BPW

    for j in range(NCH):
        pltpu.sync_copy(user.at[pl.ds(base + j * L, L)], uidx.at[j])
        pltpu.sync_copy(item.at[pl.ds(base + j * L, L)], iidx.at[j])
        upair[j] = uidx[j] >> 1
        ipair[j] = iidx[j] >> 1
    pltpu.sync_copy(gbias.at[pl.ds(0, 1)], gbv.at[pl.ds(0, 1)])

    bias_copies = []
    for j in range(NCH):
        dst = pl.ds(j * L, L)
        bias_copies.append(
            pltpu.async_copy(ubias.at[uidx.at[j]], ubv.at[dst], bsem))
        bias_copies.append(
            pltpu.async_copy(ibias.at[iidx.at[j]], ibv.at[dst], bsem))

    def fire(j, parity):
        dst = pl.ds(parity * L, L)
        pltpu.async_copy(ufp.at[upair.at[j]], urows.at[dst], gsem)
        pltpu.async_copy(ifp.at[ipair.at[j]], irows.at[dst], gsem)

    fire(0, 0)
    for c in bias_copies:
        c.wait()

    gb = gbv[...][0]
    slot0 = lax.iota(jnp.int32, L)

    def group(j, carry):
        parity = jnp.bitwise_and(j, 1)
        dst = pl.ds(parity * L, L)
        pltpu.make_async_copy(ufp.at[upair.at[0]], urows.at[dst], gsem).wait()
        pltpu.make_async_copy(ifp.at[ipair.at[0]], irows.at[dst], gsem).wait()

        @pl.when(j + 1 < NCH)
        def _():
            fire(j + 1, 1 - parity)

        uhalf = jnp.bitwise_and(uidx[j], 1) * D
        ihalf = jnp.bitwise_and(iidx[j], 1) * D
        slot = slot0 + parity * L
        acc = ubv[pl.ds(j * L, L)] + ibv[pl.ds(j * L, L)] + gb
        for c in range(D):
            acc = acc + (plsc.load_gather(urows, [slot, uhalf + c])
                         * plsc.load_gather(irows, [slot, ihalf + c]))
        outv[pl.ds(j * L, L)] = acc
        return carry

    lax.fori_loop(0, NCH, group, 0)

    pltpu.sync_copy(outv, out.at[pl.ds(base, BPW)])


def kernel(user, item, user_factors, item_factors, user_biases, item_biases,
           global_bias):
    ufp = user_factors.reshape(-1, 2 * D)
    ifp = item_factors.reshape(-1, 2 * D)
    return _mf_kernel(user, item, ufp, ifp,
                      user_biases.reshape(-1), item_biases.reshape(-1),
                      global_bias)


# speedup vs baseline: 1.1557x; 1.1557x over previous
import functools

import jax
import jax.numpy as jnp
from jax import lax
from jax.experimental import pallas as pl
from jax.experimental.pallas import tpu as pltpu
from jax.experimental.pallas import tpu_sc as plsc

NC = 2
NS = 16
NW = NC * NS
L = 16
B = 16384
D = 64
N = 1000000
NBLK = 7812
TAIL0 = NBLK * 128
TAILN = N - TAIL0
BPT = 488
SB = 1
SLABW = SB * 128
SROWS = 16400
DUMP = 16384
BIG = 0x3FFFFFFF

_mesh = plsc.VectorSubcoreMesh(core_axis_name="c", subcore_axis_name="s",
                               num_cores=NC, num_subcores=NS)

_IOTA = lambda: lax.iota(jnp.int32, L)


@functools.partial(
    pl.kernel,
    out_type=(jax.ShapeDtypeStruct((SROWS, 128), jnp.float32),
              jax.ShapeDtypeStruct((SROWS, 128), jnp.float32)),
    mesh=_mesh,
    scratch_types=[
        pltpu.VMEM((B,), jnp.int32),
        pltpu.VMEM((B + L,), jnp.int32),
        pltpu.VMEM((B,), jnp.int32),
        pltpu.VMEM((2 * D, SLABW), jnp.float32),
        pltpu.VMEM((2 * SLABW,), jnp.float32),
        pltpu.VMEM((256, 128), jnp.float32),
        pltpu.VMEM((2, 128), jnp.int32),
        pltpu.VMEM((L,), jnp.int32),
        pltpu.VMEM((L,), jnp.int32),
        pltpu.SemaphoreType.DMA,
        pltpu.SemaphoreType.DMA,
        pltpu.SemaphoreType.DMA,
    ],
    compiler_params=pltpu.CompilerParams(needs_layout_passes=False),
)
def _extract_kernel(user, item, uft, ift, ubias, ibias,
                    tail_uf, tail_if, tail_ub, tail_ib, uvals, ivals,
                    idxall, rlist, plist, slab, bslab, staging, pchunk,
                    tmpr, tmpp, ssem, bsem, csem):
    core = lax.axis_index("c")
    tid = lax.axis_index("s")
    lane0 = _IOTA() == 0

    for j in range(2):
        for k in range(128 // L):
            pchunk[j, pl.ds(k * L, L)] = jnp.full((L,), DUMP, jnp.int32)

    @pl.when(core == 0)
    def _():
        _extract_one(user, uft, ubias, tail_uf, tail_ub, uvals, tid,
                     idxall, rlist, plist, slab, bslab, staging, pchunk,
                     tmpr, tmpp, ssem, bsem, csem, lane0)

    @pl.when(core == 1)
    def _():
        _extract_one(item, ift, ibias, tail_if, tail_ib, ivals, tid,
                     idxall, rlist, plist, slab, bslab, staging, pchunk,
                     tmpr, tmpp, ssem, bsem, csem, lane0)


def _extract_one(bidx, tbl, bias, tail_t, tail_b, vals, tid,
                 idxall, rlist, plist, slab, bslab, staging, pchunk,
                 tmpr, tmpp, ssem, bsem, csem, lane0):
    b0 = tid * BPT
    last = tid == NS - 1
    nsub = jnp.where(last, NBLK - (NS - 1) * BPT, BPT)
    r_lo = b0 * 128
    r_hi = jnp.where(last, TAIL0, r_lo + BPT * 128)

    pltpu.sync_copy(bidx, idxall)

    def filt(g, off):
        rv = idxall[pl.ds(g * L, L)]
        m = (rv >= r_lo) & (rv < r_hi)
        cnt = plsc.all_reduce_population_count(m)[0]
        plsc.store_compressed(rlist.at[pl.ds(off, L)], rv, mask=m)
        pv = _IOTA() + g * L
        plsc.store_compressed(plist.at[pl.ds(off, L)], pv, mask=m)
        return off + cnt

    n_w = lax.fori_loop(0, B // L, filt, 0)
    rlist[pl.ds(n_w, L)] = jnp.full((L,), BIG, jnp.int32)
    ng = (n_w + L - 1) // L

    def fire(s, par):
        c0 = (b0 + s * SB) * 128
        pltpu.async_copy(tbl.at[:, pl.ds(c0, SLABW)],
                         slab.at[pl.ds(par * D, D)], ssem)
        pltpu.async_copy(bias.at[pl.ds(c0, SLABW)],
                         bslab.at[pl.ds(par * SLABW, SLABW)], bsem)

    def wait(par):
        pltpu.make_async_copy(tbl.at[:, pl.ds(0, SLABW)],
                              slab.at[pl.ds(par * D, D)], ssem).wait()
        pltpu.make_async_copy(bias.at[pl.ds(0, SLABW)],
                              bslab.at[pl.ds(par * SLABW, SLABW)], bsem).wait()

    def flush(cpar):
        pltpu.async_copy(staging.at[pl.ds(cpar * 128, 128)],
                         vals.at[pchunk.at[cpar]], csem)

    def drain_chunk(cpar):
        pltpu.make_async_copy(staging.at[pl.ds(cpar * 128, 128)],
                              vals.at[pchunk.at[cpar]], csem).wait()

    def do_match(t, carry, sub_lo, spar):
        sc = carry
        slot = lax.rem(sc, 128)
        cpar = lax.rem(sc // 128, 2)

        @pl.when((slot == 0) & (sc >= 256))
        def _():
            drain_chunk(cpar)
            for k in range(128 // L):
                pchunk[cpar, pl.ds(k * L, L)] = jnp.full((L,), DUMP,
                                                         jnp.int32)

        r_s = plsc.load_gather(tmpr, [jnp.full((L,), t, jnp.int32)])[0]
        p_s = plsc.load_gather(tmpp, [jnp.full((L,), t, jnp.int32)])[0]
        off = jnp.full((L,), r_s - sub_lo, jnp.int32)
        row = cpar * 128 + slot
        for k in range(D // L):
            cv = _IOTA() + (spar * D + k * L)
            staging[row, pl.ds(k * L, L)] = plsc.load_gather(slab, [cv, off])
        bv = plsc.load_gather(bslab, [jnp.full((L,), spar * SLABW, jnp.int32)
                                      + off])
        staging[row, pl.ds(D, L)] = bv
        plsc.store_scatter(pchunk, [jnp.full((L,), cpar, jnp.int32),
                                    jnp.full((L,), slot, jnp.int32)],
                           jnp.full((L,), p_s, jnp.int32), mask=lane0)

        @pl.when(slot == 127)
        def _():
            flush(cpar)

        return sc + 1

    def scan_groups(sub_lo, sub_hi, spar, sc):
        def grp(g, carry):
            rv = rlist[pl.ds(g * L, L)]
            m2 = (rv >= sub_lo) & (rv < sub_hi)
            pc2 = plsc.all_reduce_population_count(m2)[0]

            def has(carry):
                plsc.store_compressed(tmpr.at[pl.ds(0, L)], rv, mask=m2)
                pv = plist[pl.ds(g * L, L)]
                plsc.store_compressed(tmpp.at[pl.ds(0, L)], pv, mask=m2)
                return lax.fori_loop(
                    0, pc2, lambda t, c: do_match(t, c, sub_lo, spar), carry)

            return lax.cond(pc2 > 0, has, lambda c: c, carry)

        return lax.fori_loop(0, ng, grp, sc)

    fire(0, 0)

    def subslab(s, sc):
        par = lax.rem(s, 2)
        wait(par)

        @pl.when(s + 1 < nsub)
        def _():
            fire(s + 1, 1 - par)

        sub_lo = (b0 + s * SB) * 128
        return scan_groups(sub_lo, sub_lo + SLABW, par, sc)

    sc = lax.fori_loop(0, nsub, subslab, 0)

    def tail(sc):
        pltpu.sync_copy(tail_t, slab.at[pl.ds(0, D), pl.ds(0, 128)])
        pltpu.sync_copy(tail_b, bslab.at[pl.ds(0, 128)])
        return scan_groups(TAIL0, N, 0, sc)

    sc = lax.cond(tid == NS - 1, tail, lambda c: c, sc)

    @pl.when(lax.rem(sc, 128) > 0)
    def _():
        flush(lax.rem(sc // 128, 2))

    nfl = (sc + 127) // 128

    @pl.when(nfl >= 2)
    def _():
        drain_chunk(0)

    @pl.when(nfl >= 1)
    def _():
        drain_chunk(1)


@functools.partial(
    pl.kernel,
    out_type=jax.ShapeDtypeStruct((B,), jnp.float32),
    mesh=_mesh,
    scratch_types=[
        pltpu.VMEM((256, 128), jnp.float32),
        pltpu.VMEM((256, 128), jnp.float32),
        pltpu.VMEM((B // NW,), jnp.float32),
        pltpu.VMEM((L,), jnp.float32),
        pltpu.SemaphoreType.DMA,
    ],
    compiler_params=pltpu.CompilerParams(needs_layout_passes=False),
)
def _dot_kernel(uvals, ivals, gbias, out, ubuf, ibuf, outv, gbv, sem):
    wid = lax.axis_index("s") * NC + lax.axis_index("c")
    base = wid * (B // NW)
    pltpu.sync_copy(gbias.at[pl.ds(0, 1)], gbv.at[pl.ds(0, 1)])
    gb = gbv[...][0]
    slot0 = _IOTA()

    def fire(j, par):
        src = pl.ds(base + j * 128, 128)
        dst = pl.ds(par * 128, 128)
        pltpu.async_copy(uvals.at[src], ubuf.at[dst], sem)
        pltpu.async_copy(ivals.at[src], ibuf.at[dst], sem)

    def wait(par):
        dst = pl.ds(par * 128, 128)
        pltpu.make_async_copy(uvals.at[pl.ds(0, 128)], ubuf.at[dst],
                              sem).wait()
        pltpu.make_async_copy(ivals.at[pl.ds(0, 128)], ibuf.at[dst],
                              sem).wait()

    fire(0, 0)

    def chunk(j, carry):
        par = lax.rem(j, 2)
        wait(par)

        @pl.when(j + 1 < 4)
        def _():
            fire(j + 1, 1 - par)

        def group(g, c2):
            slot = par * 128 + g * L + slot0
            acc = (plsc.load_gather(ubuf, [slot, jnp.full((L,), D, jnp.int32)])
                   + plsc.load_gather(ibuf,
                                      [slot, jnp.full((L,), D, jnp.int32)])
                   + gb)
            for c in range(D):
                cv = jnp.full((L,), c, jnp.int32)
                acc = acc + (plsc.load_gather(ubuf, [slot, cv])
                             * plsc.load_gather(ibuf, [slot, cv]))
            outv[pl.ds(j * 128 + g * L, L)] = acc
            return c2

        return lax.fori_loop(0, 128 // L, group, carry)

    lax.fori_loop(0, 4, chunk, 0)
    pltpu.sync_copy(outv, out.at[pl.ds(base, B // NW)])


def kernel(user, item, user_factors, item_factors, user_biases, item_biases,
           global_bias):
    uft = user_factors.T
    ift = item_factors.T
    tail_uf = jnp.pad(user_factors[TAIL0:].T, ((0, 0), (0, 128 - TAILN)))
    tail_if = jnp.pad(item_factors[TAIL0:].T, ((0, 0), (0, 128 - TAILN)))
    tail_ub = jnp.pad(user_biases[TAIL0:, 0], (0, 128 - TAILN))
    tail_ib = jnp.pad(item_biases[TAIL0:, 0], (0, 128 - TAILN))
    uvals, ivals = _extract_kernel(user, item, uft, ift,
                                   user_biases.reshape(-1),
                                   item_biases.reshape(-1),
                                   tail_uf, tail_if, tail_ub, tail_ib)
    return _dot_kernel(uvals, ivals, global_bias)


# speedup vs baseline: 1.6743x; 1.4488x over previous
import functools

import jax
import jax.numpy as jnp
from jax import lax
from jax.experimental import pallas as pl
from jax.experimental.pallas import tpu as pltpu
from jax.experimental.pallas import tpu_sc as plsc

NC = 2
NS = 16
NW = NC * NS
L = 16
B = 16384
D = 64
N = 1000000
NBLK = 7812
TAIL0 = NBLK * 128
TAILN = N - TAIL0
BPT = 488
SB = 1
SLABW = SB * 128
SROWS = 16400
DUMP = 16384
BIG = 0x3FFFFFFF
BPB = 16
NBIN = 31

_mesh = plsc.VectorSubcoreMesh(core_axis_name="c", subcore_axis_name="s",
                               num_cores=NC, num_subcores=NS)

_IOTA = lambda: lax.iota(jnp.int32, L)


@functools.partial(
    pl.kernel,
    out_type=(jax.ShapeDtypeStruct((SROWS, 128), jnp.float32),
              jax.ShapeDtypeStruct((SROWS, 128), jnp.float32)),
    mesh=_mesh,
    scratch_types=[
        pltpu.VMEM((B,), jnp.int32),
        pltpu.VMEM((B + L,), jnp.int32),
        pltpu.VMEM((B + L,), jnp.int32),
        pltpu.VMEM((NBIN + L,), jnp.int32),
        pltpu.VMEM((2 * D, SLABW), jnp.float32),
        pltpu.VMEM((2 * SLABW,), jnp.float32),
        pltpu.VMEM((256, 128), jnp.float32),
        pltpu.VMEM((2, 128), jnp.int32),
        pltpu.VMEM((L,), jnp.int32),
        pltpu.SemaphoreType.DMA,
        pltpu.SemaphoreType.DMA,
        pltpu.SemaphoreType.DMA,
    ],
    compiler_params=pltpu.CompilerParams(needs_layout_passes=False),
)
def _extract_kernel(user, item, uft, ift, ubias, ibias,
                    tail_uf, tail_if, tail_ub, tail_ib, uvals, ivals,
                    idxall, rlist, rbin, binoff,
                    slab, bslab, staging, pchunk,
                    tmpr, ssem, bsem, csem):
    core = lax.axis_index("c")
    tid = lax.axis_index("s")
    lane0 = _IOTA() == 0

    for j in range(2):
        for k in range(128 // L):
            pchunk[j, pl.ds(k * L, L)] = jnp.full((L,), DUMP, jnp.int32)

    @pl.when(core == 0)
    def _():
        _extract_one(user, uft, ubias, tail_uf, tail_ub, uvals, tid,
                     idxall, rlist, rbin, binoff,
                     slab, bslab, staging, pchunk,
                     tmpr, ssem, bsem, csem, lane0)

    @pl.when(core == 1)
    def _():
        _extract_one(item, ift, ibias, tail_if, tail_ib, ivals, tid,
                     idxall, rlist, rbin, binoff,
                     slab, bslab, staging, pchunk,
                     tmpr, ssem, bsem, csem, lane0)


def _extract_one(bidx, tbl, bias, tail_t, tail_b, vals, tid,
                 idxall, rlist, rbin, binoff,
                 slab, bslab, staging, pchunk,
                 tmpr, ssem, bsem, csem, lane0):
    b0 = tid * BPT
    last = tid == NS - 1
    nsub = jnp.where(last, NBLK - (NS - 1) * BPT, BPT)
    r_lo = b0 * 128
    r_hi = jnp.where(last, N, r_lo + BPT * 128)

    pltpu.sync_copy(bidx, idxall)

    def filt(g, off):
        rv = idxall[pl.ds(g * L, L)]
        m = (rv >= r_lo) & (rv < r_hi)
        cnt = plsc.all_reduce_population_count(m)[0]
        pk = ((rv - r_lo) << 14) | (_IOTA() + g * L)
        plsc.store_compressed(rlist.at[pl.ds(off, L)], pk, mask=m)
        return off + cnt

    n_w = lax.fori_loop(0, B // L, filt, 0)
    rlist[pl.ds(n_w, L)] = jnp.full((L,), BIG, jnp.int32)
    ng = (n_w + L - 1) // L

    def bin_one(b, off2):
        bin_lo = (b * BPB * 128) << 14
        bin_hi = ((b + 1) * BPB * 128) << 14

        def bg(g, o2):
            pk = rlist[pl.ds(g * L, L)]
            m = (pk >= bin_lo) & (pk < bin_hi)
            cnt = plsc.all_reduce_population_count(m)[0]
            plsc.store_compressed(rbin.at[pl.ds(o2, L)], pk, mask=m)
            return o2 + cnt

        off2 = lax.fori_loop(0, ng, bg, off2)
        plsc.store_scatter(binoff, [jnp.full((L,), b + 1, jnp.int32)],
                           jnp.full((L,), off2, jnp.int32), mask=lane0)
        return off2

    plsc.store_scatter(binoff, [jnp.zeros((L,), jnp.int32)],
                       jnp.zeros((L,), jnp.int32), mask=lane0)
    off2 = 0
    for b in range(NBIN):
        off2 = bin_one(b, off2)
    rbin[pl.ds(n_w, L)] = jnp.full((L,), BIG, jnp.int32)

    def fire(s, par):
        c0 = (b0 + s * SB) * 128
        pltpu.async_copy(tbl.at[:, pl.ds(c0, SLABW)],
                         slab.at[pl.ds(par * D, D)], ssem)
        pltpu.async_copy(bias.at[pl.ds(c0, SLABW)],
                         bslab.at[pl.ds(par * SLABW, SLABW)], bsem)

    def wait(par):
        pltpu.make_async_copy(tbl.at[:, pl.ds(0, SLABW)],
                              slab.at[pl.ds(par * D, D)], ssem).wait()
        pltpu.make_async_copy(bias.at[pl.ds(0, SLABW)],
                              bslab.at[pl.ds(par * SLABW, SLABW)], bsem).wait()

    def flush(cpar):
        pltpu.async_copy(staging.at[pl.ds(cpar * 128, 128)],
                         vals.at[pchunk.at[cpar]], csem)

    def drain_chunk(cpar):
        pltpu.make_async_copy(staging.at[pl.ds(cpar * 128, 128)],
                              vals.at[pchunk.at[cpar]], csem).wait()

    def do_match(t, carry, sub_rel, spar):
        sc = carry
        slot = lax.rem(sc, 128)
        cpar = lax.rem(sc // 128, 2)

        @pl.when((slot == 0) & (sc >= 128))
        def _():
            drain_chunk(1 - cpar)
            for k in range(128 // L):
                pchunk[cpar, pl.ds(k * L, L)] = jnp.full((L,), DUMP,
                                                         jnp.int32)

        pk_s = plsc.load_gather(tmpr, [jnp.full((L,), t, jnp.int32)])[0]
        p_s = pk_s & 0x3FFF
        off = jnp.full((L,), (pk_s >> 14) - sub_rel, jnp.int32)
        row = cpar * 128 + slot
        for k in range(D // L):
            cv = _IOTA() + (spar * D + k * L)
            staging[row, pl.ds(k * L, L)] = plsc.load_gather(slab, [cv, off])
        bv = plsc.load_gather(bslab, [jnp.full((L,), spar * SLABW, jnp.int32)
                                      + off])
        staging[row, pl.ds(D, L)] = bv
        plsc.store_scatter(pchunk, [jnp.full((L,), cpar, jnp.int32),
                                    jnp.full((L,), slot, jnp.int32)],
                           jnp.full((L,), p_s, jnp.int32), mask=lane0)

        @pl.when(slot == 127)
        def _():
            flush(cpar)

        return sc + 1

    def scan_groups(sub_rel, sub_rel_hi, spar, sc, bst, bend):
        def grp(g, carry):
            pk = rbin[pl.ds(bst + g * L, L)]
            m2 = (pk >= (sub_rel << 14)) & (pk < (sub_rel_hi << 14))
            pc2 = plsc.all_reduce_population_count(m2)[0]

            def has(carry):
                plsc.store_compressed(tmpr.at[pl.ds(0, L)], pk, mask=m2)
                return lax.fori_loop(
                    0, pc2, lambda t, c: do_match(t, c, sub_rel, spar),
                    carry)

            return lax.cond(pc2 > 0, has, lambda c: c, carry)

        ng2 = (bend - bst + L - 1) // L
        return lax.fori_loop(0, ng2, grp, sc)

    def bin_bounds_aligned(b):
        bst, bend = bin_bounds(b)
        return bst & ~(L - 1), bend

    def bin_bounds(b):
        bst = plsc.load_gather(binoff, [jnp.full((L,), b, jnp.int32)])[0]
        bend = plsc.load_gather(binoff,
                                [jnp.full((L,), b + 1, jnp.int32)])[0]
        return bst, bend

    fire(0, 0)

    def subslab(s, sc):
        par = lax.rem(s, 2)
        wait(par)

        @pl.when(s + 1 < nsub)
        def _():
            fire(s + 1, 1 - par)

        sub_rel = s * SLABW
        bst, bend = bin_bounds_aligned(s // BPB)
        return scan_groups(sub_rel, sub_rel + SLABW, par, sc, bst, bend)

    sc = lax.fori_loop(0, nsub, subslab, 0)

    def tail(sc):
        pltpu.sync_copy(tail_t, slab.at[pl.ds(0, D), pl.ds(0, 128)])
        pltpu.sync_copy(tail_b, bslab.at[pl.ds(0, 128)])
        bst, bend = bin_bounds_aligned(NBIN - 1)
        return scan_groups(TAIL0 - r_lo, N - r_lo, 0, sc, bst, bend)

    sc = lax.cond(tid == NS - 1, tail, lambda c: c, sc)

    @pl.when(lax.rem(sc, 128) > 0)
    def _():
        flush(lax.rem(sc // 128, 2))

    @pl.when(sc > 0)
    def _():
        drain_chunk(lax.rem((sc - 1) // 128, 2))


@functools.partial(
    pl.kernel,
    out_type=jax.ShapeDtypeStruct((B,), jnp.float32),
    mesh=_mesh,
    scratch_types=[
        pltpu.VMEM((256, 128), jnp.float32),
        pltpu.VMEM((256, 128), jnp.float32),
        pltpu.VMEM((B // NW,), jnp.float32),
        pltpu.VMEM((L,), jnp.float32),
        pltpu.SemaphoreType.DMA,
    ],
    compiler_params=pltpu.CompilerParams(needs_layout_passes=False),
)
def _dot_kernel(uvals, ivals, gbias, out, ubuf, ibuf, outv, gbv, sem):
    wid = lax.axis_index("s") * NC + lax.axis_index("c")
    base = wid * (B // NW)
    pltpu.sync_copy(gbias.at[pl.ds(0, 1)], gbv.at[pl.ds(0, 1)])
    gb = gbv[...][0]
    slot0 = _IOTA()

    def fire(j, par):
        src = pl.ds(base + j * 128, 128)
        dst = pl.ds(par * 128, 128)
        pltpu.async_copy(uvals.at[src], ubuf.at[dst], sem)
        pltpu.async_copy(ivals.at[src], ibuf.at[dst], sem)

    def wait(par):
        dst = pl.ds(par * 128, 128)
        pltpu.make_async_copy(uvals.at[pl.ds(0, 128)], ubuf.at[dst],
                              sem).wait()
        pltpu.make_async_copy(ivals.at[pl.ds(0, 128)], ibuf.at[dst],
                              sem).wait()

    fire(0, 0)

    def chunk(j, carry):
        par = lax.rem(j, 2)
        wait(par)

        @pl.when(j + 1 < 4)
        def _():
            fire(j + 1, 1 - par)

        def group(g, c2):
            slot = par * 128 + g * L + slot0
            acc = (plsc.load_gather(ubuf, [slot, jnp.full((L,), D, jnp.int32)])
                   + plsc.load_gather(ibuf,
                                      [slot, jnp.full((L,), D, jnp.int32)])
                   + gb)
            for c in range(D):
                cv = jnp.full((L,), c, jnp.int32)
                acc = acc + (plsc.load_gather(ubuf, [slot, cv])
                             * plsc.load_gather(ibuf, [slot, cv]))
            outv[pl.ds(j * 128 + g * L, L)] = acc
            return c2

        return lax.fori_loop(0, 128 // L, group, carry)

    lax.fori_loop(0, 4, chunk, 0)
    pltpu.sync_copy(outv, out.at[pl.ds(base, B // NW)])


def kernel(user, item, user_factors, item_factors, user_biases, item_biases,
           global_bias):
    uft = user_factors.T
    ift = item_factors.T
    tail_uf = jnp.pad(user_factors[TAIL0:].T, ((0, 0), (0, 128 - TAILN)))
    tail_if = jnp.pad(item_factors[TAIL0:].T, ((0, 0), (0, 128 - TAILN)))
    tail_ub = jnp.pad(user_biases[TAIL0:, 0], (0, 128 - TAILN))
    tail_ib = jnp.pad(item_biases[TAIL0:, 0], (0, 128 - TAILN))
    uvals, ivals = _extract_kernel(user, item, uft, ift,
                                   user_biases.reshape(-1),
                                   item_biases.reshape(-1),
                                   tail_uf, tail_if, tail_ub, tail_ib)
    return _dot_kernel(uvals, ivals, global_bias)


# speedup vs baseline: 2.7306x; 1.6309x over previous
import functools

import jax
import jax.numpy as jnp
from jax import lax
from jax.experimental import pallas as pl
from jax.experimental.pallas import tpu as pltpu
from jax.experimental.pallas import tpu_sc as plsc

NC = 2
NS = 16
NW = NC * NS
L = 16
B = 16384
D = 64
N = 1000000
NBLK = 7812
TAIL0 = NBLK * 128
TAILN = N - TAIL0
BPT = 488
SB = 1
SLABW = SB * 128
SROWS = 16400
DUMP = 16384
BIG = 0x3FFFFFFF
BPB = 16
NBIN = 31
NBUF = 4

_mesh = plsc.VectorSubcoreMesh(core_axis_name="c", subcore_axis_name="s",
                               num_cores=NC, num_subcores=NS)

_IOTA = lambda: lax.iota(jnp.int32, L)


@functools.partial(
    pl.kernel,
    out_type=(jax.ShapeDtypeStruct((SROWS, 128), jnp.float32),
              jax.ShapeDtypeStruct((SROWS, 128), jnp.float32)),
    mesh=_mesh,
    scratch_types=[
        pltpu.VMEM((B,), jnp.int32),
        pltpu.VMEM((B + L,), jnp.int32),
        pltpu.VMEM((B + L,), jnp.int32),
        pltpu.VMEM((NBIN + L,), jnp.int32),
        pltpu.VMEM((NBUF * D, SLABW), jnp.float32),
        pltpu.VMEM((NBUF * SLABW,), jnp.float32),
        pltpu.VMEM((256, 128), jnp.float32),
        pltpu.VMEM((2, 128), jnp.int32),
        pltpu.VMEM((L,), jnp.int32),
        pltpu.SemaphoreType.DMA,
        pltpu.SemaphoreType.DMA,
        pltpu.SemaphoreType.DMA,
    ],
    compiler_params=pltpu.CompilerParams(needs_layout_passes=False),
)
def _extract_kernel(user, item, uft, ift, ubias, ibias,
                    tail_uf, tail_if, tail_ub, tail_ib, uvals, ivals,
                    idxall, rlist, rbin, binoff,
                    slab, bslab, staging, pchunk,
                    tmpr, ssem, bsem, csem):
    core = lax.axis_index("c")
    tid = lax.axis_index("s")
    lane0 = _IOTA() == 0

    for j in range(2):
        for k in range(128 // L):
            pchunk[j, pl.ds(k * L, L)] = jnp.full((L,), DUMP, jnp.int32)

    @pl.when(core == 0)
    def _():
        _extract_one(user, uft, ubias, tail_uf, tail_ub, uvals, tid,
                     idxall, rlist, rbin, binoff,
                     slab, bslab, staging, pchunk,
                     tmpr, ssem, bsem, csem, lane0)

    @pl.when(core == 1)
    def _():
        _extract_one(item, ift, ibias, tail_if, tail_ib, ivals, tid,
                     idxall, rlist, rbin, binoff,
                     slab, bslab, staging, pchunk,
                     tmpr, ssem, bsem, csem, lane0)


def _extract_one(bidx, tbl, bias, tail_t, tail_b, vals, tid,
                 idxall, rlist, rbin, binoff,
                 slab, bslab, staging, pchunk,
                 tmpr, ssem, bsem, csem, lane0):
    b0 = tid * BPT
    last = tid == NS - 1
    nsub = jnp.where(last, NBLK - (NS - 1) * BPT, BPT)
    r_lo = b0 * 128
    r_hi = jnp.where(last, N, r_lo + BPT * 128)

    pltpu.sync_copy(bidx, idxall)

    def filt(g, off):
        rv = idxall[pl.ds(g * L, L)]
        m = (rv >= r_lo) & (rv < r_hi)
        cnt = plsc.all_reduce_population_count(m)[0]
        pk = ((rv - r_lo) << 14) | (_IOTA() + g * L)
        plsc.store_compressed(rlist.at[pl.ds(off, L)], pk, mask=m)
        return off + cnt

    n_w = lax.fori_loop(0, B // L, filt, 0)
    rlist[pl.ds(n_w, L)] = jnp.full((L,), BIG, jnp.int32)
    ng = (n_w + L - 1) // L

    def bin_one(b, off2):
        bin_lo = (b * BPB * 128) << 14
        bin_hi = ((b + 1) * BPB * 128) << 14

        def bg(g, o2):
            pk = rlist[pl.ds(g * L, L)]
            m = (pk >= bin_lo) & (pk < bin_hi)
            cnt = plsc.all_reduce_population_count(m)[0]
            plsc.store_compressed(rbin.at[pl.ds(o2, L)], pk, mask=m)
            return o2 + cnt

        off2 = lax.fori_loop(0, ng, bg, off2)
        plsc.store_scatter(binoff, [jnp.full((L,), b + 1, jnp.int32)],
                           jnp.full((L,), off2, jnp.int32), mask=lane0)
        return off2

    plsc.store_scatter(binoff, [jnp.zeros((L,), jnp.int32)],
                       jnp.zeros((L,), jnp.int32), mask=lane0)
    off2 = 0
    for b in range(NBIN):
        off2 = bin_one(b, off2)
    rbin[pl.ds(n_w, L)] = jnp.full((L,), BIG, jnp.int32)

    def fire(s, par):
        c0 = (b0 + s * SB) * 128
        pltpu.async_copy(tbl.at[:, pl.ds(c0, SLABW)],
                         slab.at[pl.ds(par * D, D)], ssem)
        pltpu.async_copy(bias.at[pl.ds(c0, SLABW)],
                         bslab.at[pl.ds(par * SLABW, SLABW)], bsem)

    def wait(par):
        pltpu.make_async_copy(tbl.at[:, pl.ds(0, SLABW)],
                              slab.at[pl.ds(par * D, D)], ssem).wait()
        pltpu.make_async_copy(bias.at[pl.ds(0, SLABW)],
                              bslab.at[pl.ds(par * SLABW, SLABW)], bsem).wait()

    def flush(cpar):
        pltpu.async_copy(staging.at[pl.ds(cpar * 128, 128)],
                         vals.at[pchunk.at[cpar]], csem)

    def drain_chunk(cpar):
        pltpu.make_async_copy(staging.at[pl.ds(cpar * 128, 128)],
                              vals.at[pchunk.at[cpar]], csem).wait()

    def do_match(t, carry, sub_rel, spar):
        sc = carry
        slot = lax.rem(sc, 128)
        cpar = lax.rem(sc // 128, 2)

        @pl.when((slot == 0) & (sc >= 128))
        def _():
            drain_chunk(1 - cpar)
            for k in range(128 // L):
                pchunk[cpar, pl.ds(k * L, L)] = jnp.full((L,), DUMP,
                                                         jnp.int32)

        pk_s = plsc.load_gather(tmpr, [jnp.full((L,), t, jnp.int32)])[0]
        p_s = pk_s & 0x3FFF
        off = jnp.full((L,), (pk_s >> 14) - sub_rel, jnp.int32)
        row = cpar * 128 + slot
        for k in range(D // L):
            cv = _IOTA() + (spar * D + k * L)
            staging[row, pl.ds(k * L, L)] = plsc.load_gather(slab, [cv, off])
        bv = plsc.load_gather(bslab, [jnp.full((L,), spar * SLABW, jnp.int32)
                                      + off])
        staging[row, pl.ds(D, L)] = bv
        plsc.store_scatter(pchunk, [jnp.full((L,), cpar, jnp.int32),
                                    jnp.full((L,), slot, jnp.int32)],
                           jnp.full((L,), p_s, jnp.int32), mask=lane0)

        @pl.when(slot == 127)
        def _():
            flush(cpar)

        return sc + 1

    def scan_groups(sub_rel, sub_rel_hi, spar, sc, bst, bend):
        def grp(g, carry):
            pk = rbin[pl.ds(bst + g * L, L)]
            m2 = (pk >= (sub_rel << 14)) & (pk < (sub_rel_hi << 14))
            pc2 = plsc.all_reduce_population_count(m2)[0]

            def has(carry):
                plsc.store_compressed(tmpr.at[pl.ds(0, L)], pk, mask=m2)
                return lax.fori_loop(
                    0, pc2, lambda t, c: do_match(t, c, sub_rel, spar),
                    carry)

            return lax.cond(pc2 > 0, has, lambda c: c, carry)

        ng2 = (bend - bst + L - 1) // L
        return lax.fori_loop(0, ng2, grp, sc)

    def bin_bounds_aligned(b):
        bst, bend = bin_bounds(b)
        return bst & ~(L - 1), bend

    def bin_bounds(b):
        bst = plsc.load_gather(binoff, [jnp.full((L,), b, jnp.int32)])[0]
        bend = plsc.load_gather(binoff,
                                [jnp.full((L,), b + 1, jnp.int32)])[0]
        return bst, bend

    for p in range(NBUF - 1):
        fire(p, p)

    def subslab(s, sc):
        par = lax.rem(s, NBUF)
        wait(par)

        @pl.when(s + NBUF - 1 < nsub)
        def _():
            fire(s + NBUF - 1, lax.rem(s + NBUF - 1, NBUF))

        sub_rel = s * SLABW
        bst, bend = bin_bounds_aligned(s // BPB)
        return scan_groups(sub_rel, sub_rel + SLABW, par, sc, bst, bend)

    sc = lax.fori_loop(0, nsub, subslab, 0)

    def tail(sc):
        pltpu.sync_copy(tail_t, slab.at[pl.ds(0, D), pl.ds(0, 128)])
        pltpu.sync_copy(tail_b, bslab.at[pl.ds(0, 128)])
        bst, bend = bin_bounds_aligned(NBIN - 1)
        return scan_groups(TAIL0 - r_lo, N - r_lo, 0, sc, bst, bend)

    sc = lax.cond(tid == NS - 1, tail, lambda c: c, sc)

    @pl.when(lax.rem(sc, 128) > 0)
    def _():
        flush(lax.rem(sc // 128, 2))

    @pl.when(sc > 0)
    def _():
        drain_chunk(lax.rem((sc - 1) // 128, 2))


@functools.partial(
    pl.kernel,
    out_type=jax.ShapeDtypeStruct((B,), jnp.float32),
    mesh=_mesh,
    scratch_types=[
        pltpu.VMEM((256, 128), jnp.float32),
        pltpu.VMEM((256, 128), jnp.float32),
        pltpu.VMEM((B // NW,), jnp.float32),
        pltpu.VMEM((L,), jnp.float32),
        pltpu.SemaphoreType.DMA,
    ],
    compiler_params=pltpu.CompilerParams(needs_layout_passes=False),
)
def _dot_kernel(uvals, ivals, gbias, out, ubuf, ibuf, outv, gbv, sem):
    wid = lax.axis_index("s") * NC + lax.axis_index("c")
    base = wid * (B // NW)
    pltpu.sync_copy(gbias.at[pl.ds(0, 1)], gbv.at[pl.ds(0, 1)])
    gb = gbv[...][0]
    slot0 = _IOTA()

    def fire(j, par):
        src = pl.ds(base + j * 128, 128)
        dst = pl.ds(par * 128, 128)
        pltpu.async_copy(uvals.at[src], ubuf.at[dst], sem)
        pltpu.async_copy(ivals.at[src], ibuf.at[dst], sem)

    def wait(par):
        dst = pl.ds(par * 128, 128)
        pltpu.make_async_copy(uvals.at[pl.ds(0, 128)], ubuf.at[dst],
                              sem).wait()
        pltpu.make_async_copy(ivals.at[pl.ds(0, 128)], ibuf.at[dst],
                              sem).wait()

    fire(0, 0)

    def chunk(j, carry):
        par = lax.rem(j, 2)
        wait(par)

        @pl.when(j + 1 < 4)
        def _():
            fire(j + 1, 1 - par)

        def group(g, c2):
            slot = par * 128 + g * L + slot0
            acc = (plsc.load_gather(ubuf, [slot, jnp.full((L,), D, jnp.int32)])
                   + plsc.load_gather(ibuf,
                                      [slot, jnp.full((L,), D, jnp.int32)])
                   + gb)
            for c in range(D):
                cv = jnp.full((L,), c, jnp.int32)
                acc = acc + (plsc.load_gather(ubuf, [slot, cv])
                             * plsc.load_gather(ibuf, [slot, cv]))
            outv[pl.ds(j * 128 + g * L, L)] = acc
            return c2

        return lax.fori_loop(0, 128 // L, group, carry)

    lax.fori_loop(0, 4, chunk, 0)
    pltpu.sync_copy(outv, out.at[pl.ds(base, B // NW)])


def kernel(user, item, user_factors, item_factors, user_biases, item_biases,
           global_bias):
    uft = user_factors.T
    ift = item_factors.T
    tail_uf = jnp.pad(user_factors[TAIL0:].T, ((0, 0), (0, 128 - TAILN)))
    tail_if = jnp.pad(item_factors[TAIL0:].T, ((0, 0), (0, 128 - TAILN)))
    tail_ub = jnp.pad(user_biases[TAIL0:, 0], (0, 128 - TAILN))
    tail_ib = jnp.pad(item_biases[TAIL0:, 0], (0, 128 - TAILN))
    uvals, ivals = _extract_kernel(user, item, uft, ift,
                                   user_biases.reshape(-1),
                                   item_biases.reshape(-1),
                                   tail_uf, tail_if, tail_ub, tail_ib)
    return _dot_kernel(uvals, ivals, global_bias)


# speedup vs baseline: 2.9169x; 1.0682x over previous
import functools

import jax
import jax.numpy as jnp
from jax import lax
from jax.experimental import pallas as pl
from jax.experimental.pallas import tpu as pltpu
from jax.experimental.pallas import tpu_sc as plsc

NC = 2
NS = 16
NW = NC * NS
L = 16
B = 16384
D = 64
N = 1000000
NBLK = 7812
TAIL0 = NBLK * 128
TAILN = N - TAIL0
BPT = 488
SB = 1
SLABW = SB * 128
SROWS = 16400
DUMP = 16384
BIG = 0x3FFFFFFF
BPB = 16
NBIN = 31
NBUF = 6
IDXC = 2048

_mesh = plsc.VectorSubcoreMesh(core_axis_name="c", subcore_axis_name="s",
                               num_cores=NC, num_subcores=NS)

_IOTA = lambda: lax.iota(jnp.int32, L)


@functools.partial(
    pl.kernel,
    out_type=(jax.ShapeDtypeStruct((SROWS, 128), jnp.float32),
              jax.ShapeDtypeStruct((SROWS, 128), jnp.float32)),
    mesh=_mesh,
    scratch_types=[
        pltpu.VMEM((IDXC,), jnp.int32),
        pltpu.VMEM((B + L,), jnp.int32),
        pltpu.VMEM((B + L,), jnp.int32),
        pltpu.VMEM((NBIN + L,), jnp.int32),
        pltpu.VMEM((NBUF * D, SLABW), jnp.float32),
        pltpu.VMEM((NBUF * SLABW,), jnp.float32),
        pltpu.VMEM((256, 128), jnp.float32),
        pltpu.VMEM((2, 128), jnp.int32),
        pltpu.VMEM((L,), jnp.int32),
        pltpu.SemaphoreType.DMA,
        pltpu.SemaphoreType.DMA,
        pltpu.SemaphoreType.DMA,
    ],
    compiler_params=pltpu.CompilerParams(needs_layout_passes=False),
)
def _extract_kernel(user, item, uft, ift, ubias, ibias,
                    tail_uf, tail_if, tail_ub, tail_ib, uvals, ivals,
                    idxall, rlist, rbin, binoff,
                    slab, bslab, staging, pchunk,
                    tmpr, ssem, bsem, csem):
    core = lax.axis_index("c")
    tid = lax.axis_index("s")
    lane0 = _IOTA() == 0

    for j in range(2):
        for k in range(128 // L):
            pchunk[j, pl.ds(k * L, L)] = jnp.full((L,), DUMP, jnp.int32)

    @pl.when(core == 0)
    def _():
        _extract_one(user, uft, ubias, tail_uf, tail_ub, uvals, tid,
                     idxall, rlist, rbin, binoff,
                     slab, bslab, staging, pchunk,
                     tmpr, ssem, bsem, csem, lane0)

    @pl.when(core == 1)
    def _():
        _extract_one(item, ift, ibias, tail_if, tail_ib, ivals, tid,
                     idxall, rlist, rbin, binoff,
                     slab, bslab, staging, pchunk,
                     tmpr, ssem, bsem, csem, lane0)


def _extract_one(bidx, tbl, bias, tail_t, tail_b, vals, tid,
                 idxall, rlist, rbin, binoff,
                 slab, bslab, staging, pchunk,
                 tmpr, ssem, bsem, csem, lane0):
    b0 = tid * BPT
    last = tid == NS - 1
    nsub = jnp.where(last, NBLK - (NS - 1) * BPT, BPT)
    r_lo = b0 * 128
    r_hi = jnp.where(last, N, r_lo + BPT * 128)

    n_w = 0
    for c in range(B // IDXC):
        pltpu.sync_copy(bidx.at[pl.ds(c * IDXC, IDXC)], idxall)

        def filt(g, off, c=c):
            rv = idxall[pl.ds(g * L, L)]
            m = (rv >= r_lo) & (rv < r_hi)
            cnt = plsc.all_reduce_population_count(m)[0]
            pk = ((rv - r_lo) << 14) | (_IOTA() + (c * IDXC + g * L))
            plsc.store_compressed(rlist.at[pl.ds(off, L)], pk, mask=m)
            return off + cnt

        n_w = lax.fori_loop(0, IDXC // L, filt, n_w)
    rlist[pl.ds(n_w, L)] = jnp.full((L,), BIG, jnp.int32)
    ng = (n_w + L - 1) // L

    def bin_one(b, off2):
        bin_lo = (b * BPB * 128) << 14
        bin_hi = ((b + 1) * BPB * 128) << 14

        def bg(g, o2):
            pk = rlist[pl.ds(g * L, L)]
            m = (pk >= bin_lo) & (pk < bin_hi)
            cnt = plsc.all_reduce_population_count(m)[0]
            plsc.store_compressed(rbin.at[pl.ds(o2, L)], pk, mask=m)
            return o2 + cnt

        off2 = lax.fori_loop(0, ng, bg, off2)
        plsc.store_scatter(binoff, [jnp.full((L,), b + 1, jnp.int32)],
                           jnp.full((L,), off2, jnp.int32), mask=lane0)
        return off2

    plsc.store_scatter(binoff, [jnp.zeros((L,), jnp.int32)],
                       jnp.zeros((L,), jnp.int32), mask=lane0)
    off2 = 0
    for b in range(NBIN):
        off2 = bin_one(b, off2)
    rbin[pl.ds(n_w, L)] = jnp.full((L,), BIG, jnp.int32)

    def fire(s, par):
        c0 = (b0 + s * SB) * 128
        pltpu.async_copy(tbl.at[:, pl.ds(c0, SLABW)],
                         slab.at[pl.ds(par * D, D)], ssem)
        pltpu.async_copy(bias.at[pl.ds(c0, SLABW)],
                         bslab.at[pl.ds(par * SLABW, SLABW)], bsem)

    def wait(par):
        pltpu.make_async_copy(tbl.at[:, pl.ds(0, SLABW)],
                              slab.at[pl.ds(par * D, D)], ssem).wait()
        pltpu.make_async_copy(bias.at[pl.ds(0, SLABW)],
                              bslab.at[pl.ds(par * SLABW, SLABW)], bsem).wait()

    def flush(cpar):
        pltpu.async_copy(staging.at[pl.ds(cpar * 128, 128)],
                         vals.at[pchunk.at[cpar]], csem)

    def drain_chunk(cpar):
        pltpu.make_async_copy(staging.at[pl.ds(cpar * 128, 128)],
                              vals.at[pchunk.at[cpar]], csem).wait()

    def do_match(t, carry, sub_rel, spar):
        sc = carry
        slot = lax.rem(sc, 128)
        cpar = lax.rem(sc // 128, 2)

        @pl.when((slot == 0) & (sc >= 128))
        def _():
            drain_chunk(1 - cpar)
            for k in range(128 // L):
                pchunk[cpar, pl.ds(k * L, L)] = jnp.full((L,), DUMP,
                                                         jnp.int32)

        pk_s = plsc.load_gather(tmpr, [jnp.full((L,), t, jnp.int32)])[0]
        p_s = pk_s & 0x3FFF
        off = jnp.full((L,), (pk_s >> 14) - sub_rel, jnp.int32)
        row = cpar * 128 + slot
        for k in range(D // L):
            cv = _IOTA() + (spar * D + k * L)
            staging[row, pl.ds(k * L, L)] = plsc.load_gather(slab, [cv, off])
        bv = plsc.load_gather(bslab, [jnp.full((L,), spar * SLABW, jnp.int32)
                                      + off])
        staging[row, pl.ds(D, L)] = bv
        plsc.store_scatter(pchunk, [jnp.full((L,), cpar, jnp.int32),
                                    jnp.full((L,), slot, jnp.int32)],
                           jnp.full((L,), p_s, jnp.int32), mask=lane0)

        @pl.when(slot == 127)
        def _():
            flush(cpar)

        return sc + 1

    def scan_groups(sub_rel, sub_rel_hi, spar, sc, bst, bend):
        def grp(g, carry):
            pk = rbin[pl.ds(bst + g * L, L)]
            m2 = (pk >= (sub_rel << 14)) & (pk < (sub_rel_hi << 14))
            pc2 = plsc.all_reduce_population_count(m2)[0]

            def has(carry):
                plsc.store_compressed(tmpr.at[pl.ds(0, L)], pk, mask=m2)
                return lax.fori_loop(
                    0, pc2, lambda t, c: do_match(t, c, sub_rel, spar),
                    carry)

            return lax.cond(pc2 > 0, has, lambda c: c, carry)

        ng2 = (bend - bst + L - 1) // L
        return lax.fori_loop(0, ng2, grp, sc)

    def bin_bounds_aligned(b):
        bst, bend = bin_bounds(b)
        return bst & ~(L - 1), bend

    def bin_bounds(b):
        bst = plsc.load_gather(binoff, [jnp.full((L,), b, jnp.int32)])[0]
        bend = plsc.load_gather(binoff,
                                [jnp.full((L,), b + 1, jnp.int32)])[0]
        return bst, bend

    for p in range(NBUF - 1):
        fire(p, p)

    def subslab(s, sc):
        par = lax.rem(s, NBUF)
        wait(par)

        @pl.when(s + NBUF - 1 < nsub)
        def _():
            fire(s + NBUF - 1, lax.rem(s + NBUF - 1, NBUF))

        sub_rel = s * SLABW
        bst, bend = bin_bounds_aligned(s // BPB)
        return scan_groups(sub_rel, sub_rel + SLABW, par, sc, bst, bend)

    sc = lax.fori_loop(0, nsub, subslab, 0)

    def tail(sc):
        pltpu.sync_copy(tail_t, slab.at[pl.ds(0, D), pl.ds(0, 128)])
        pltpu.sync_copy(tail_b, bslab.at[pl.ds(0, 128)])
        bst, bend = bin_bounds_aligned(NBIN - 1)
        return scan_groups(TAIL0 - r_lo, N - r_lo, 0, sc, bst, bend)

    sc = lax.cond(tid == NS - 1, tail, lambda c: c, sc)

    @pl.when(lax.rem(sc, 128) > 0)
    def _():
        flush(lax.rem(sc // 128, 2))

    @pl.when(sc > 0)
    def _():
        drain_chunk(lax.rem((sc - 1) // 128, 2))


@functools.partial(
    pl.kernel,
    out_type=jax.ShapeDtypeStruct((B,), jnp.float32),
    mesh=_mesh,
    scratch_types=[
        pltpu.VMEM((256, 128), jnp.float32),
        pltpu.VMEM((256, 128), jnp.float32),
        pltpu.VMEM((B // NW,), jnp.float32),
        pltpu.VMEM((L,), jnp.float32),
        pltpu.SemaphoreType.DMA,
    ],
    compiler_params=pltpu.CompilerParams(needs_layout_passes=False),
)
def _dot_kernel(uvals, ivals, gbias, out, ubuf, ibuf, outv, gbv, sem):
    wid = lax.axis_index("s") * NC + lax.axis_index("c")
    base = wid * (B // NW)
    pltpu.sync_copy(gbias.at[pl.ds(0, 1)], gbv.at[pl.ds(0, 1)])
    gb = gbv[...][0]
    slot0 = _IOTA()

    def fire(j, par):
        src = pl.ds(base + j * 128, 128)
        dst = pl.ds(par * 128, 128)
        pltpu.async_copy(uvals.at[src], ubuf.at[dst], sem)
        pltpu.async_copy(ivals.at[src], ibuf.at[dst], sem)

    def wait(par):
        dst = pl.ds(par * 128, 128)
        pltpu.make_async_copy(uvals.at[pl.ds(0, 128)], ubuf.at[dst],
                              sem).wait()
        pltpu.make_async_copy(ivals.at[pl.ds(0, 128)], ibuf.at[dst],
                              sem).wait()

    fire(0, 0)

    def chunk(j, carry):
        par = lax.rem(j, 2)
        wait(par)

        @pl.when(j + 1 < 4)
        def _():
            fire(j + 1, 1 - par)

        def group(g, c2):
            slot = par * 128 + g * L + slot0
            acc = (plsc.load_gather(ubuf, [slot, jnp.full((L,), D, jnp.int32)])
                   + plsc.load_gather(ibuf,
                                      [slot, jnp.full((L,), D, jnp.int32)])
                   + gb)
            for c in range(D):
                cv = jnp.full((L,), c, jnp.int32)
                acc = acc + (plsc.load_gather(ubuf, [slot, cv])
                             * plsc.load_gather(ibuf, [slot, cv]))
            outv[pl.ds(j * 128 + g * L, L)] = acc
            return c2

        return lax.fori_loop(0, 128 // L, group, carry)

    lax.fori_loop(0, 4, chunk, 0)
    pltpu.sync_copy(outv, out.at[pl.ds(base, B // NW)])


def kernel(user, item, user_factors, item_factors, user_biases, item_biases,
           global_bias):
    uft = user_factors.T
    ift = item_factors.T
    tail_uf = jnp.pad(user_factors[TAIL0:].T, ((0, 0), (0, 128 - TAILN)))
    tail_if = jnp.pad(item_factors[TAIL0:].T, ((0, 0), (0, 128 - TAILN)))
    tail_ub = jnp.pad(user_biases[TAIL0:, 0], (0, 128 - TAILN))
    tail_ib = jnp.pad(item_biases[TAIL0:, 0], (0, 128 - TAILN))
    uvals, ivals = _extract_kernel(user, item, uft, ift,
                                   user_biases.reshape(-1),
                                   item_biases.reshape(-1),
                                   tail_uf, tail_if, tail_ub, tail_ib)
    return _dot_kernel(uvals, ivals, global_bias)


# speedup vs baseline: 2.9248x; 1.0027x over previous
import functools

import jax
import jax.numpy as jnp
from jax import lax
from jax.experimental import pallas as pl
from jax.experimental.pallas import tpu as pltpu
from jax.experimental.pallas import tpu_sc as plsc

NC = 2
NS = 16
NW = NC * NS
L = 16
B = 16384
D = 64
N = 1000000
NBLK = 7812
TAIL0 = NBLK * 128
TAILN = N - TAIL0
BPT = 488
SB = 1
SLABW = SB * 128
SROWS = 16400
DUMP = 16384
BIG = 0x3FFFFFFF
BPB = 16
NBIN = 31
NBUF = 6
IDXC = 2048

_mesh = plsc.VectorSubcoreMesh(core_axis_name="c", subcore_axis_name="s",
                               num_cores=NC, num_subcores=NS)

_IOTA = lambda: lax.iota(jnp.int32, L)


@functools.partial(
    pl.kernel,
    out_type=(jax.ShapeDtypeStruct((SROWS, 128), jnp.float32),
              jax.ShapeDtypeStruct((SROWS, 128), jnp.float32)),
    mesh=_mesh,
    scratch_types=[
        pltpu.VMEM((IDXC,), jnp.int32),
        pltpu.VMEM((B + L,), jnp.int32),
        pltpu.VMEM((B + L,), jnp.int32),
        pltpu.VMEM((NBIN + L,), jnp.int32),
        pltpu.VMEM((NBUF * D, SLABW), jnp.float32),
        pltpu.VMEM((NBUF * SLABW,), jnp.float32),
        pltpu.VMEM((256, 128), jnp.float32),
        pltpu.VMEM((2, 128), jnp.int32),
        pltpu.VMEM((L,), jnp.int32),
        pltpu.SemaphoreType.DMA,
        pltpu.SemaphoreType.DMA,
        pltpu.SemaphoreType.DMA,
    ],
    compiler_params=pltpu.CompilerParams(needs_layout_passes=False),
)
def _extract_kernel(user, item, uft, ift, ubias, ibias,
                    tail_uf, tail_if, tail_ub, tail_ib, uvals, ivals,
                    idxall, rlist, rbin, binoff,
                    slab, bslab, staging, pchunk,
                    tmpr, ssem, bsem, csem):
    core = lax.axis_index("c")
    tid = lax.axis_index("s")
    lane0 = _IOTA() == 0

    for j in range(2):
        for k in range(128 // L):
            pchunk[j, pl.ds(k * L, L)] = jnp.full((L,), DUMP, jnp.int32)

    @pl.when(core == 0)
    def _():
        _extract_one(user, uft, ubias, tail_uf, tail_ub, uvals, tid,
                     idxall, rlist, rbin, binoff,
                     slab, bslab, staging, pchunk,
                     tmpr, ssem, bsem, csem, lane0)

    @pl.when(core == 1)
    def _():
        _extract_one(item, ift, ibias, tail_if, tail_ib, ivals, tid,
                     idxall, rlist, rbin, binoff,
                     slab, bslab, staging, pchunk,
                     tmpr, ssem, bsem, csem, lane0)


def _extract_one(bidx, tbl, bias, tail_t, tail_b, vals, tid,
                 idxall, rlist, rbin, binoff,
                 slab, bslab, staging, pchunk,
                 tmpr, ssem, bsem, csem, lane0):
    b0 = tid * BPT
    last = tid == NS - 1
    nsub = jnp.where(last, NBLK - (NS - 1) * BPT, BPT)
    r_lo = b0 * 128
    r_hi = jnp.where(last, N, r_lo + BPT * 128)

    n_w = 0
    for c in range(B // IDXC):
        pltpu.sync_copy(bidx.at[pl.ds(c * IDXC, IDXC)], idxall)

        def filt(g, off, c=c):
            rv = idxall[pl.ds(g * L, L)]
            m = (rv >= r_lo) & (rv < r_hi)
            cnt = plsc.all_reduce_population_count(m)[0]
            pk = ((rv - r_lo) << 14) | (_IOTA() + (c * IDXC + g * L))
            plsc.store_compressed(rlist.at[pl.ds(off, L)], pk, mask=m)
            return off + cnt

        n_w = lax.fori_loop(0, IDXC // L, filt, n_w)
    rlist[pl.ds(n_w, L)] = jnp.full((L,), BIG, jnp.int32)
    ng = (n_w + L - 1) // L

    def bin_one(b, off2):
        bin_lo = (b * BPB * 128) << 14
        bin_hi = ((b + 1) * BPB * 128) << 14

        def bg(g, o2):
            pk = rlist[pl.ds(g * L, L)]
            m = (pk >= bin_lo) & (pk < bin_hi)
            cnt = plsc.all_reduce_population_count(m)[0]
            plsc.store_compressed(rbin.at[pl.ds(o2, L)], pk, mask=m)
            return o2 + cnt

        off2 = lax.fori_loop(0, ng, bg, off2)
        plsc.store_scatter(binoff, [jnp.full((L,), b + 1, jnp.int32)],
                           jnp.full((L,), off2, jnp.int32), mask=lane0)
        return off2

    plsc.store_scatter(binoff, [jnp.zeros((L,), jnp.int32)],
                       jnp.zeros((L,), jnp.int32), mask=lane0)
    off2 = 0
    for b in range(NBIN):
        off2 = bin_one(b, off2)
    rbin[pl.ds(n_w, L)] = jnp.full((L,), BIG, jnp.int32)

    def fire(s, par):
        c0 = (b0 + s * SB) * 128
        pltpu.async_copy(tbl.at[:, pl.ds(c0, SLABW)],
                         slab.at[pl.ds(par * D, D)], ssem)
        pltpu.async_copy(bias.at[pl.ds(c0, SLABW)],
                         bslab.at[pl.ds(par * SLABW, SLABW)], bsem)

    def wait(par):
        pltpu.make_async_copy(tbl.at[:, pl.ds(0, SLABW)],
                              slab.at[pl.ds(par * D, D)], ssem).wait()
        pltpu.make_async_copy(bias.at[pl.ds(0, SLABW)],
                              bslab.at[pl.ds(par * SLABW, SLABW)], bsem).wait()

    def flush(cpar):
        pltpu.async_copy(staging.at[pl.ds(cpar * 128, 128)],
                         vals.at[pchunk.at[cpar]], csem)

    def drain_chunk(cpar):
        pltpu.make_async_copy(staging.at[pl.ds(cpar * 128, 128)],
                              vals.at[pchunk.at[cpar]], csem).wait()

    def do_match(t, carry, sub_rel, spar):
        sc = carry
        slot = lax.rem(sc, 128)
        cpar = lax.rem(sc // 128, 2)

        @pl.when((slot == 0) & (sc >= 128))
        def _():
            drain_chunk(1 - cpar)
            for k in range(128 // L):
                pchunk[cpar, pl.ds(k * L, L)] = jnp.full((L,), DUMP,
                                                         jnp.int32)

        pk_s = plsc.load_gather(tmpr, [jnp.full((L,), t, jnp.int32)])[0]
        p_s = pk_s & 0x3FFF
        off = jnp.full((L,), (pk_s >> 14) - sub_rel, jnp.int32)
        row = cpar * 128 + slot
        for k in range(D // L):
            cv = _IOTA() + (spar * D + k * L)
            staging[row, pl.ds(k * L, L)] = plsc.load_gather(slab, [cv, off])
        bv = plsc.load_gather(bslab, [jnp.full((L,), spar * SLABW, jnp.int32)
                                      + off])
        staging[row, pl.ds(D, L)] = bv
        plsc.store_scatter(pchunk, [jnp.full((L,), cpar, jnp.int32),
                                    jnp.full((L,), slot, jnp.int32)],
                           jnp.full((L,), p_s, jnp.int32), mask=lane0)

        @pl.when(slot == 127)
        def _():
            flush(cpar)

        return sc + 1

    def scan_groups(sub_rel, sub_rel_hi, spar, sc, bst, bend):
        def grp(g, carry):
            pk = rbin[pl.ds(bst + g * L, L)]
            m2 = (pk >= (sub_rel << 14)) & (pk < (sub_rel_hi << 14))
            pc2 = plsc.all_reduce_population_count(m2)[0]

            def has(carry):
                plsc.store_compressed(tmpr.at[pl.ds(0, L)], pk, mask=m2)
                return lax.fori_loop(
                    0, pc2, lambda t, c: do_match(t, c, sub_rel, spar),
                    carry)

            return lax.cond(pc2 > 0, has, lambda c: c, carry)

        ng2 = (bend - bst + L - 1) // L
        return lax.fori_loop(0, ng2, grp, sc)

    def bin_bounds_aligned(b):
        bst, bend = bin_bounds(b)
        return bst & ~(L - 1), bend

    def bin_bounds(b):
        bst = plsc.load_gather(binoff, [jnp.full((L,), b, jnp.int32)])[0]
        bend = plsc.load_gather(binoff,
                                [jnp.full((L,), b + 1, jnp.int32)])[0]
        return bst, bend

    for p in range(NBUF - 1):
        fire(p, p)

    def subslab(s, sc):
        par = lax.rem(s, NBUF)
        wait(par)

        @pl.when(s + NBUF - 1 < nsub)
        def _():
            fire(s + NBUF - 1, lax.rem(s + NBUF - 1, NBUF))

        sub_rel = s * SLABW
        bst, bend = bin_bounds_aligned(s // BPB)
        return scan_groups(sub_rel, sub_rel + SLABW, par, sc, bst, bend)

    sc = lax.fori_loop(0, nsub, subslab, 0)

    def tail(sc):
        pltpu.sync_copy(tail_t, slab.at[pl.ds(0, D), pl.ds(0, 128)])
        pltpu.sync_copy(tail_b, bslab.at[pl.ds(0, 128)])
        bst, bend = bin_bounds_aligned(NBIN - 1)
        return scan_groups(TAIL0 - r_lo, N - r_lo, 0, sc, bst, bend)

    sc = lax.cond(tid == NS - 1, tail, lambda c: c, sc)

    @pl.when(lax.rem(sc, 128) > 0)
    def _():
        flush(lax.rem(sc // 128, 2))

    @pl.when(sc > 0)
    def _():
        drain_chunk(lax.rem((sc - 1) // 128, 2))


@functools.partial(
    pl.kernel,
    out_type=jax.ShapeDtypeStruct((B,), jnp.float32),
    mesh=_mesh,
    scratch_types=[
        pltpu.VMEM((256, 128), jnp.float32),
        pltpu.VMEM((256, 128), jnp.float32),
        pltpu.VMEM((B // NW,), jnp.float32),
        pltpu.VMEM((L,), jnp.float32),
        pltpu.SemaphoreType.DMA,
    ],
    compiler_params=pltpu.CompilerParams(needs_layout_passes=False),
)
def _dot_kernel(uvals, ivals, gbias, out, ubuf, ibuf, outv, gbv, sem):
    wid = lax.axis_index("s") * NC + lax.axis_index("c")
    base = wid * (B // NW)
    pltpu.sync_copy(gbias.at[pl.ds(0, 1)], gbv.at[pl.ds(0, 1)])
    gb = gbv[...][0]
    slot0 = _IOTA()
    CH = 64
    NCHK = (B // NW) // CH
    RING = 4

    def fire(j, par):
        src = pl.ds(base + j * CH, CH)
        dst = pl.ds(par * CH, CH)
        pltpu.async_copy(uvals.at[src], ubuf.at[dst], sem)
        pltpu.async_copy(ivals.at[src], ibuf.at[dst], sem)

    def wait(par):
        dst = pl.ds(par * CH, CH)
        pltpu.make_async_copy(uvals.at[pl.ds(0, CH)], ubuf.at[dst],
                              sem).wait()
        pltpu.make_async_copy(ivals.at[pl.ds(0, CH)], ibuf.at[dst],
                              sem).wait()

    for p in range(RING - 1):
        fire(p, p)

    def chunk(j, carry):
        par = lax.rem(j, RING)
        wait(par)

        @pl.when(j + RING - 1 < NCHK)
        def _():
            fire(j + RING - 1, lax.rem(j + RING - 1, RING))

        def group(g, c2):
            slot = par * CH + g * L + slot0
            acc = (plsc.load_gather(ubuf, [slot, jnp.full((L,), D, jnp.int32)])
                   + plsc.load_gather(ibuf,
                                      [slot, jnp.full((L,), D, jnp.int32)])
                   + gb)
            for c in range(D):
                cv = jnp.full((L,), c, jnp.int32)
                acc = acc + (plsc.load_gather(ubuf, [slot, cv])
                             * plsc.load_gather(ibuf, [slot, cv]))
            outv[pl.ds(j * CH + g * L, L)] = acc
            return c2

        return lax.fori_loop(0, CH // L, group, carry)

    lax.fori_loop(0, NCHK, chunk, 0)
    pltpu.sync_copy(outv, out.at[pl.ds(base, B // NW)])


def kernel(user, item, user_factors, item_factors, user_biases, item_biases,
           global_bias):
    uft = user_factors.T
    ift = item_factors.T
    tail_uf = jnp.pad(user_factors[TAIL0:].T, ((0, 0), (0, 128 - TAILN)))
    tail_if = jnp.pad(item_factors[TAIL0:].T, ((0, 0), (0, 128 - TAILN)))
    tail_ub = jnp.pad(user_biases[TAIL0:, 0], (0, 128 - TAILN))
    tail_ib = jnp.pad(item_biases[TAIL0:, 0], (0, 128 - TAILN))
    uvals, ivals = _extract_kernel(user, item, uft, ift,
                                   user_biases.reshape(-1),
                                   item_biases.reshape(-1),
                                   tail_uf, tail_if, tail_ub, tail_ib)
    return _dot_kernel(uvals, ivals, global_bias)


# speedup vs baseline: 2.9747x; 1.0170x over previous
import functools

import jax
import jax.numpy as jnp
from jax import lax
from jax.experimental import pallas as pl
from jax.experimental.pallas import tpu as pltpu
from jax.experimental.pallas import tpu_sc as plsc

NC = 2
NS = 16
NW = NC * NS
L = 16
B = 16384
D = 64
N = 1000000
NBLK = 7812
TAIL0 = NBLK * 128
TAILN = N - TAIL0
BPT = 488
SB = 1
SLABW = SB * 128
SROWS = 16400
DUMP = 16384
BIG = 0x3FFFFFFF
BPB = 16
NBIN = 31
NBUF = 6
IDXC = 2048

_mesh = plsc.VectorSubcoreMesh(core_axis_name="c", subcore_axis_name="s",
                               num_cores=NC, num_subcores=NS)

_IOTA = lambda: lax.iota(jnp.int32, L)


@functools.partial(
    pl.kernel,
    out_type=(jax.ShapeDtypeStruct((B,), jnp.float32),
              jax.ShapeDtypeStruct((SROWS, 128), jnp.float32),
              jax.ShapeDtypeStruct((SROWS, 128), jnp.float32)),
    mesh=_mesh,
    scratch_types=[
        pltpu.VMEM((IDXC,), jnp.int32),
        pltpu.VMEM((B + L,), jnp.int32),
        pltpu.VMEM((B + L,), jnp.int32),
        pltpu.VMEM((NBIN + L,), jnp.int32),
        pltpu.VMEM((NBUF * D, SLABW), jnp.float32),
        pltpu.VMEM((NBUF * SLABW,), jnp.float32),
        pltpu.VMEM((256, 128), jnp.float32),
        pltpu.VMEM((2, 128), jnp.int32),
        pltpu.VMEM((L,), jnp.int32),
        pltpu.VMEM((B // NW,), jnp.float32),
        pltpu.VMEM((L,), jnp.float32),
        pltpu.SemaphoreType.DMA,
        pltpu.SemaphoreType.DMA,
        pltpu.SemaphoreType.DMA,
        pltpu.SemaphoreType.REGULAR,
    ],
    compiler_params=pltpu.CompilerParams(needs_layout_passes=False),
)
def _fused_kernel(user, item, uft, ift, ubias, ibias,
                  tail_uf, tail_if, tail_ub, tail_ib, gbias,
                  out, uvals, ivals,
                  idxall, rlist, rbin, binoff,
                  slab, bslab, staging, pchunk,
                  tmpr, outv, gbv, ssem, bsem, csem, barsem):
    core = lax.axis_index("c")
    tid = lax.axis_index("s")
    lane0 = _IOTA() == 0

    for j in range(2):
        for k in range(128 // L):
            pchunk[j, pl.ds(k * L, L)] = jnp.full((L,), DUMP, jnp.int32)

    @pl.when(core == 0)
    def _():
        _extract_one(user, uft, ubias, tail_uf, tail_ub, uvals, tid,
                     idxall, rlist, rbin, binoff,
                     slab, bslab, staging, pchunk,
                     tmpr, ssem, bsem, csem, lane0)

    @pl.when(core == 1)
    def _():
        _extract_one(item, ift, ibias, tail_if, tail_ib, ivals, tid,
                     idxall, rlist, rbin, binoff,
                     slab, bslab, staging, pchunk,
                     tmpr, ssem, bsem, csem, lane0)

    plsc.subcore_barrier()
    pl.semaphore_signal(barsem, 1, core_index=1 - core)
    pl.semaphore_wait(barsem, 1)

    wid = tid * NC + core
    base = wid * (B // NW)
    pltpu.sync_copy(gbias.at[pl.ds(0, 1)], gbv.at[pl.ds(0, 1)])
    gb = gbv[...][0]
    slot0 = _IOTA()
    CH = 64
    NCHK = (B // NW) // CH
    RING = 2

    def dfire(j, par):
        src = pl.ds(base + j * CH, CH)
        pltpu.async_copy(uvals.at[src], staging.at[pl.ds(par * CH, CH)],
                         ssem)
        pltpu.async_copy(ivals.at[src],
                         staging.at[pl.ds(128 + par * CH, CH)], bsem)

    def dwait(par):
        pltpu.make_async_copy(uvals.at[pl.ds(0, CH)],
                              staging.at[pl.ds(par * CH, CH)], ssem).wait()
        pltpu.make_async_copy(ivals.at[pl.ds(0, CH)],
                              staging.at[pl.ds(128 + par * CH, CH)],
                              bsem).wait()

    for p in range(RING - 1):
        dfire(p, p)

    def chunk(j, carry):
        par = lax.rem(j, RING)
        dwait(par)

        @pl.when(j + RING - 1 < NCHK)
        def _():
            dfire(j + RING - 1, lax.rem(j + RING - 1, RING))

        def group(g, c2):
            su = par * CH + g * L + slot0
            si = su + 128
            dcol = jnp.full((L,), D, jnp.int32)
            acc = (plsc.load_gather(staging, [su, dcol])
                   + plsc.load_gather(staging, [si, dcol]) + gb)
            for c in range(D):
                cv = jnp.full((L,), c, jnp.int32)
                acc = acc + (plsc.load_gather(staging, [su, cv])
                             * plsc.load_gather(staging, [si, cv]))
            outv[pl.ds(j * CH + g * L, L)] = acc
            return c2

        return lax.fori_loop(0, CH // L, group, carry)

    lax.fori_loop(0, NCHK, chunk, 0)
    pltpu.sync_copy(outv, out.at[pl.ds(base, B // NW)])


def _extract_one(bidx, tbl, bias, tail_t, tail_b, vals, tid,
                 idxall, rlist, rbin, binoff,
                 slab, bslab, staging, pchunk,
                 tmpr, ssem, bsem, csem, lane0):
    b0 = tid * BPT
    last = tid == NS - 1
    nsub = jnp.where(last, NBLK - (NS - 1) * BPT, BPT)
    r_lo = b0 * 128
    r_hi = jnp.where(last, N, r_lo + BPT * 128)

    n_w = 0
    for c in range(B // IDXC):
        pltpu.sync_copy(bidx.at[pl.ds(c * IDXC, IDXC)], idxall)

        def filt(g, off, c=c):
            rv = idxall[pl.ds(g * L, L)]
            m = (rv >= r_lo) & (rv < r_hi)
            cnt = plsc.all_reduce_population_count(m)[0]
            pk = ((rv - r_lo) << 14) | (_IOTA() + (c * IDXC + g * L))
            plsc.store_compressed(rlist.at[pl.ds(off, L)], pk, mask=m)
            return off + cnt

        n_w = lax.fori_loop(0, IDXC // L, filt, n_w)
    rlist[pl.ds(n_w, L)] = jnp.full((L,), BIG, jnp.int32)
    ng = (n_w + L - 1) // L

    def bin_one(b, off2):
        bin_lo = (b * BPB * 128) << 14
        bin_hi = ((b + 1) * BPB * 128) << 14

        def bg(g, o2):
            pk = rlist[pl.ds(g * L, L)]
            m = (pk >= bin_lo) & (pk < bin_hi)
            cnt = plsc.all_reduce_population_count(m)[0]
            plsc.store_compressed(rbin.at[pl.ds(o2, L)], pk, mask=m)
            return o2 + cnt

        off2 = lax.fori_loop(0, ng, bg, off2)
        plsc.store_scatter(binoff, [jnp.full((L,), b + 1, jnp.int32)],
                           jnp.full((L,), off2, jnp.int32), mask=lane0)
        return off2

    plsc.store_scatter(binoff, [jnp.zeros((L,), jnp.int32)],
                       jnp.zeros((L,), jnp.int32), mask=lane0)
    off2 = 0
    for b in range(NBIN):
        off2 = bin_one(b, off2)
    rbin[pl.ds(n_w, L)] = jnp.full((L,), BIG, jnp.int32)

    def fire(s, par):
        c0 = (b0 + s * SB) * 128
        pltpu.async_copy(tbl.at[:, pl.ds(c0, SLABW)],
                         slab.at[pl.ds(par * D, D)], ssem)
        pltpu.async_copy(bias.at[pl.ds(c0, SLABW)],
                         bslab.at[pl.ds(par * SLABW, SLABW)], bsem)

    def wait(par):
        pltpu.make_async_copy(tbl.at[:, pl.ds(0, SLABW)],
                              slab.at[pl.ds(par * D, D)], ssem).wait()
        pltpu.make_async_copy(bias.at[pl.ds(0, SLABW)],
                              bslab.at[pl.ds(par * SLABW, SLABW)], bsem).wait()

    def flush(cpar):
        pltpu.async_copy(staging.at[pl.ds(cpar * 128, 128)],
                         vals.at[pchunk.at[cpar]], csem)

    def drain_chunk(cpar):
        pltpu.make_async_copy(staging.at[pl.ds(cpar * 128, 128)],
                              vals.at[pchunk.at[cpar]], csem).wait()

    def do_match(t, carry, sub_rel, spar):
        sc = carry
        slot = lax.rem(sc, 128)
        cpar = lax.rem(sc // 128, 2)

        @pl.when((slot == 0) & (sc >= 128))
        def _():
            drain_chunk(1 - cpar)
            for k in range(128 // L):
                pchunk[cpar, pl.ds(k * L, L)] = jnp.full((L,), DUMP,
                                                         jnp.int32)

        pk_s = plsc.load_gather(tmpr, [jnp.full((L,), t, jnp.int32)])[0]
        p_s = pk_s & 0x3FFF
        off = jnp.full((L,), (pk_s >> 14) - sub_rel, jnp.int32)
        row = cpar * 128 + slot
        for k in range(D // L):
            cv = _IOTA() + (spar * D + k * L)
            staging[row, pl.ds(k * L, L)] = plsc.load_gather(slab, [cv, off])
        bv = plsc.load_gather(bslab, [jnp.full((L,), spar * SLABW, jnp.int32)
                                      + off])
        staging[row, pl.ds(D, L)] = bv
        plsc.store_scatter(pchunk, [jnp.full((L,), cpar, jnp.int32),
                                    jnp.full((L,), slot, jnp.int32)],
                           jnp.full((L,), p_s, jnp.int32), mask=lane0)

        @pl.when(slot == 127)
        def _():
            flush(cpar)

        return sc + 1

    def scan_groups(sub_rel, sub_rel_hi, spar, sc, bst, bend):
        def grp(g, carry):
            pk = rbin[pl.ds(bst + g * L, L)]
            m2 = (pk >= (sub_rel << 14)) & (pk < (sub_rel_hi << 14))
            pc2 = plsc.all_reduce_population_count(m2)[0]

            def has(carry):
                plsc.store_compressed(tmpr.at[pl.ds(0, L)], pk, mask=m2)
                return lax.fori_loop(
                    0, pc2, lambda t, c: do_match(t, c, sub_rel, spar),
                    carry)

            return lax.cond(pc2 > 0, has, lambda c: c, carry)

        ng2 = (bend - bst + L - 1) // L
        return lax.fori_loop(0, ng2, grp, sc)

    def bin_bounds_aligned(b):
        bst, bend = bin_bounds(b)
        return bst & ~(L - 1), bend

    def bin_bounds(b):
        bst = plsc.load_gather(binoff, [jnp.full((L,), b, jnp.int32)])[0]
        bend = plsc.load_gather(binoff,
                                [jnp.full((L,), b + 1, jnp.int32)])[0]
        return bst, bend

    for p in range(NBUF - 1):
        fire(p, p)

    def subslab(s, sc):
        par = lax.rem(s, NBUF)
        wait(par)

        @pl.when(s + NBUF - 1 < nsub)
        def _():
            fire(s + NBUF - 1, lax.rem(s + NBUF - 1, NBUF))

        sub_rel = s * SLABW
        bst, bend = bin_bounds_aligned(s // BPB)
        return scan_groups(sub_rel, sub_rel + SLABW, par, sc, bst, bend)

    sc = lax.fori_loop(0, nsub, subslab, 0)

    def tail(sc):
        pltpu.sync_copy(tail_t, slab.at[pl.ds(0, D), pl.ds(0, 128)])
        pltpu.sync_copy(tail_b, bslab.at[pl.ds(0, 128)])
        bst, bend = bin_bounds_aligned(NBIN - 1)
        return scan_groups(TAIL0 - r_lo, N - r_lo, 0, sc, bst, bend)

    sc = lax.cond(tid == NS - 1, tail, lambda c: c, sc)

    @pl.when(lax.rem(sc, 128) > 0)
    def _():
        flush(lax.rem(sc // 128, 2))

    @pl.when(sc > 0)
    def _():
        drain_chunk(lax.rem((sc - 1) // 128, 2))


def kernel(user, item, user_factors, item_factors, user_biases, item_biases,
           global_bias):
    uft = user_factors.T
    ift = item_factors.T
    tail_uf = jnp.pad(user_factors[TAIL0:].T, ((0, 0), (0, 128 - TAILN)))
    tail_if = jnp.pad(item_factors[TAIL0:].T, ((0, 0), (0, 128 - TAILN)))
    tail_ub = jnp.pad(user_biases[TAIL0:, 0], (0, 128 - TAILN))
    tail_ib = jnp.pad(item_biases[TAIL0:, 0], (0, 128 - TAILN))
    out, _, _ = _fused_kernel(user, item, uft, ift,
                              user_biases.reshape(-1),
                              item_biases.reshape(-1),
                              tail_uf, tail_if, tail_ub, tail_ib,
                              global_bias)
    return out


# speedup vs baseline: 2.9934x; 1.0063x over previous
import functools

import jax
import jax.numpy as jnp
from jax import lax
from jax.experimental import pallas as pl
from jax.experimental.pallas import tpu as pltpu
from jax.experimental.pallas import tpu_sc as plsc

NC = 2
NS = 16
NW = NC * NS
L = 16
B = 16384
D = 64
N = 1000000
NBLK = 7812
TAIL0 = NBLK * 128
TAILN = N - TAIL0
BPT = 488
SB = 1
SLABW = SB * 128
SROWS = 16400
DUMP = 16384
BIG = 0x3FFFFFFF
BPB = 16
NBIN = 31
NBUF = 7
IDXC = 2048

_mesh = plsc.VectorSubcoreMesh(core_axis_name="c", subcore_axis_name="s",
                               num_cores=NC, num_subcores=NS)

_IOTA = lambda: lax.iota(jnp.int32, L)


@functools.partial(
    pl.kernel,
    out_type=(jax.ShapeDtypeStruct((B,), jnp.float32),
              jax.ShapeDtypeStruct((SROWS, 128), jnp.float32),
              jax.ShapeDtypeStruct((SROWS, 128), jnp.float32)),
    mesh=_mesh,
    scratch_types=[
        pltpu.VMEM((IDXC,), jnp.int32),
        pltpu.VMEM((B + L,), jnp.int32),
        pltpu.VMEM((B + L,), jnp.int32),
        pltpu.VMEM((NBIN + L,), jnp.int32),
        pltpu.VMEM((NBUF * D, SLABW), jnp.float32),
        pltpu.VMEM((NBUF * SLABW,), jnp.float32),
        pltpu.VMEM((256, 128), jnp.float32),
        pltpu.VMEM((2, 128), jnp.int32),
        pltpu.VMEM((L,), jnp.int32),
        pltpu.VMEM((B // NW,), jnp.float32),
        pltpu.VMEM((L,), jnp.float32),
        pltpu.SemaphoreType.DMA,
        pltpu.SemaphoreType.DMA,
        pltpu.SemaphoreType.DMA,
        pltpu.SemaphoreType.REGULAR,
    ],
    compiler_params=pltpu.CompilerParams(needs_layout_passes=False),
)
def _fused_kernel(user, item, uft, ift, ubias, ibias,
                  tail_uf, tail_if, tail_ub, tail_ib, gbias,
                  out, uvals, ivals,
                  idxall, rlist, rbin, binoff,
                  slab, bslab, staging, pchunk,
                  tmpr, outv, gbv, ssem, bsem, csem, barsem):
    core = lax.axis_index("c")
    tid = lax.axis_index("s")
    lane0 = _IOTA() == 0

    for j in range(2):
        for k in range(128 // L):
            pchunk[j, pl.ds(k * L, L)] = jnp.full((L,), DUMP, jnp.int32)

    @pl.when(core == 0)
    def _():
        _extract_one(user, uft, ubias, tail_uf, tail_ub, uvals, tid,
                     idxall, rlist, rbin, binoff,
                     slab, bslab, staging, pchunk,
                     tmpr, ssem, bsem, csem, lane0)

    @pl.when(core == 1)
    def _():
        _extract_one(item, ift, ibias, tail_if, tail_ib, ivals, tid,
                     idxall, rlist, rbin, binoff,
                     slab, bslab, staging, pchunk,
                     tmpr, ssem, bsem, csem, lane0)

    plsc.subcore_barrier()
    pl.semaphore_signal(barsem, 1, core_index=1 - core)
    pl.semaphore_wait(barsem, 1)

    wid = tid * NC + core
    base = wid * (B // NW)
    pltpu.sync_copy(gbias.at[pl.ds(0, 1)], gbv.at[pl.ds(0, 1)])
    gb = gbv[...][0]
    slot0 = _IOTA()
    CH = 64
    NCHK = (B // NW) // CH
    RING = 2

    def dfire(j, par):
        src = pl.ds(base + j * CH, CH)
        pltpu.async_copy(uvals.at[src], staging.at[pl.ds(par * CH, CH)],
                         ssem)
        pltpu.async_copy(ivals.at[src],
                         staging.at[pl.ds(128 + par * CH, CH)], bsem)

    def dwait(par):
        pltpu.make_async_copy(uvals.at[pl.ds(0, CH)],
                              staging.at[pl.ds(par * CH, CH)], ssem).wait()
        pltpu.make_async_copy(ivals.at[pl.ds(0, CH)],
                              staging.at[pl.ds(128 + par * CH, CH)],
                              bsem).wait()

    for p in range(RING - 1):
        dfire(p, p)

    def chunk(j, carry):
        par = lax.rem(j, RING)
        dwait(par)

        @pl.when(j + RING - 1 < NCHK)
        def _():
            dfire(j + RING - 1, lax.rem(j + RING - 1, RING))

        def group(g, c2):
            su = par * CH + g * L + slot0
            si = su + 128
            dcol = jnp.full((L,), D, jnp.int32)
            acc = (plsc.load_gather(staging, [su, dcol])
                   + plsc.load_gather(staging, [si, dcol]) + gb)
            for c in range(D):
                cv = jnp.full((L,), c, jnp.int32)
                acc = acc + (plsc.load_gather(staging, [su, cv])
                             * plsc.load_gather(staging, [si, cv]))
            outv[pl.ds(j * CH + g * L, L)] = acc
            return c2

        return lax.fori_loop(0, CH // L, group, carry)

    lax.fori_loop(0, NCHK, chunk, 0)
    pltpu.sync_copy(outv, out.at[pl.ds(base, B // NW)])


def _extract_one(bidx, tbl, bias, tail_t, tail_b, vals, tid,
                 idxall, rlist, rbin, binoff,
                 slab, bslab, staging, pchunk,
                 tmpr, ssem, bsem, csem, lane0):
    b0 = tid * BPT
    last = tid == NS - 1
    nsub = jnp.where(last, NBLK - (NS - 1) * BPT, BPT)
    r_lo = b0 * 128
    r_hi = jnp.where(last, N, r_lo + BPT * 128)

    n_w = 0
    for c in range(B // IDXC):
        pltpu.sync_copy(bidx.at[pl.ds(c * IDXC, IDXC)], idxall)

        def filt(g, off, c=c):
            rv = idxall[pl.ds(g * L, L)]
            m = (rv >= r_lo) & (rv < r_hi)
            cnt = plsc.all_reduce_population_count(m)[0]
            pk = ((rv - r_lo) << 14) | (_IOTA() + (c * IDXC + g * L))
            plsc.store_compressed(rlist.at[pl.ds(off, L)], pk, mask=m)
            return off + cnt

        n_w = lax.fori_loop(0, IDXC // L, filt, n_w)
    rlist[pl.ds(n_w, L)] = jnp.full((L,), BIG, jnp.int32)
    ng = (n_w + L - 1) // L

    def bin_one(b, off2):
        bin_lo = (b * BPB * 128) << 14
        bin_hi = ((b + 1) * BPB * 128) << 14

        def bg(g, o2):
            pk = rlist[pl.ds(g * L, L)]
            m = (pk >= bin_lo) & (pk < bin_hi)
            cnt = plsc.all_reduce_population_count(m)[0]
            plsc.store_compressed(rbin.at[pl.ds(o2, L)], pk, mask=m)
            return o2 + cnt

        off2 = lax.fori_loop(0, ng, bg, off2)
        plsc.store_scatter(binoff, [jnp.full((L,), b + 1, jnp.int32)],
                           jnp.full((L,), off2, jnp.int32), mask=lane0)
        return off2

    plsc.store_scatter(binoff, [jnp.zeros((L,), jnp.int32)],
                       jnp.zeros((L,), jnp.int32), mask=lane0)
    off2 = 0
    for b in range(NBIN):
        off2 = bin_one(b, off2)
    rbin[pl.ds(n_w, L)] = jnp.full((L,), BIG, jnp.int32)

    def fire(s, par):
        c0 = (b0 + s * SB) * 128
        pltpu.async_copy(tbl.at[:, pl.ds(c0, SLABW)],
                         slab.at[pl.ds(par * D, D)], ssem)
        pltpu.async_copy(bias.at[pl.ds(c0, SLABW)],
                         bslab.at[pl.ds(par * SLABW, SLABW)], bsem)

    def wait(par):
        pltpu.make_async_copy(tbl.at[:, pl.ds(0, SLABW)],
                              slab.at[pl.ds(par * D, D)], ssem).wait()
        pltpu.make_async_copy(bias.at[pl.ds(0, SLABW)],
                              bslab.at[pl.ds(par * SLABW, SLABW)], bsem).wait()

    def flush(cpar):
        pltpu.async_copy(staging.at[pl.ds(cpar * 128, 128)],
                         vals.at[pchunk.at[cpar]], csem)

    def drain_chunk(cpar):
        pltpu.make_async_copy(staging.at[pl.ds(cpar * 128, 128)],
                              vals.at[pchunk.at[cpar]], csem).wait()

    def do_match(t, carry, sub_rel, spar):
        sc = carry
        slot = lax.rem(sc, 128)
        cpar = lax.rem(sc // 128, 2)

        @pl.when((slot == 0) & (sc >= 128))
        def _():
            drain_chunk(1 - cpar)
            for k in range(128 // L):
                pchunk[cpar, pl.ds(k * L, L)] = jnp.full((L,), DUMP,
                                                         jnp.int32)

        pk_s = plsc.load_gather(tmpr, [jnp.full((L,), t, jnp.int32)])[0]
        p_s = pk_s & 0x3FFF
        off = jnp.full((L,), (pk_s >> 14) - sub_rel, jnp.int32)
        row = cpar * 128 + slot
        for k in range(D // L):
            cv = _IOTA() + (spar * D + k * L)
            staging[row, pl.ds(k * L, L)] = plsc.load_gather(slab, [cv, off])
        bv = plsc.load_gather(bslab, [jnp.full((L,), spar * SLABW, jnp.int32)
                                      + off])
        staging[row, pl.ds(D, L)] = bv
        plsc.store_scatter(pchunk, [jnp.full((L,), cpar, jnp.int32),
                                    jnp.full((L,), slot, jnp.int32)],
                           jnp.full((L,), p_s, jnp.int32), mask=lane0)

        @pl.when(slot == 127)
        def _():
            flush(cpar)

        return sc + 1

    def scan_groups(sub_rel, sub_rel_hi, spar, sc, bst, bend):
        def grp(g, carry):
            pk = rbin[pl.ds(bst + g * L, L)]
            m2 = (pk >= (sub_rel << 14)) & (pk < (sub_rel_hi << 14))
            pc2 = plsc.all_reduce_population_count(m2)[0]

            def has(carry):
                plsc.store_compressed(tmpr.at[pl.ds(0, L)], pk, mask=m2)
                return lax.fori_loop(
                    0, pc2, lambda t, c: do_match(t, c, sub_rel, spar),
                    carry)

            return lax.cond(pc2 > 0, has, lambda c: c, carry)

        ng2 = (bend - bst + L - 1) // L
        return lax.fori_loop(0, ng2, grp, sc)

    def bin_bounds_aligned(b):
        bst, bend = bin_bounds(b)
        return bst & ~(L - 1), bend

    def bin_bounds(b):
        bst = plsc.load_gather(binoff, [jnp.full((L,), b, jnp.int32)])[0]
        bend = plsc.load_gather(binoff,
                                [jnp.full((L,), b + 1, jnp.int32)])[0]
        return bst, bend

    for p in range(NBUF - 1):
        fire(p, p)

    def subslab(s, sc):
        par = lax.rem(s, NBUF)
        wait(par)

        @pl.when(s + NBUF - 1 < nsub)
        def _():
            fire(s + NBUF - 1, lax.rem(s + NBUF - 1, NBUF))

        sub_rel = s * SLABW
        bst, bend = bin_bounds_aligned(s // BPB)
        return scan_groups(sub_rel, sub_rel + SLABW, par, sc, bst, bend)

    sc = lax.fori_loop(0, nsub, subslab, 0)

    def tail(sc):
        pltpu.sync_copy(tail_t, slab.at[pl.ds(0, D), pl.ds(0, 128)])
        pltpu.sync_copy(tail_b, bslab.at[pl.ds(0, 128)])
        bst, bend = bin_bounds_aligned(NBIN - 1)
        return scan_groups(TAIL0 - r_lo, N - r_lo, 0, sc, bst, bend)

    sc = lax.cond(tid == NS - 1, tail, lambda c: c, sc)

    @pl.when(lax.rem(sc, 128) > 0)
    def _():
        flush(lax.rem(sc // 128, 2))

    @pl.when(sc > 0)
    def _():
        drain_chunk(lax.rem((sc - 1) // 128, 2))


def kernel(user, item, user_factors, item_factors, user_biases, item_biases,
           global_bias):
    uft = user_factors.T
    ift = item_factors.T
    tail_uf = jnp.pad(user_factors[TAIL0:].T, ((0, 0), (0, 128 - TAILN)))
    tail_if = jnp.pad(item_factors[TAIL0:].T, ((0, 0), (0, 128 - TAILN)))
    tail_ub = jnp.pad(user_biases[TAIL0:, 0], (0, 128 - TAILN))
    tail_ib = jnp.pad(item_biases[TAIL0:, 0], (0, 128 - TAILN))
    out, _, _ = _fused_kernel(user, item, uft, ift,
                              user_biases.reshape(-1),
                              item_biases.reshape(-1),
                              tail_uf, tail_if, tail_ub, tail_ib,
                              global_bias)
    return out


# speedup vs baseline: 3.0051x; 1.0039x over previous
import functools

import jax
import jax.numpy as jnp
from jax import lax
from jax.experimental import pallas as pl
from jax.experimental.pallas import tpu as pltpu
from jax.experimental.pallas import tpu_sc as plsc

NC = 2
NS = 16
NW = NC * NS
L = 16
B = 16384
D = 64
N = 1000000
NBLK = 7812
TAIL0 = NBLK * 128
TAILN = N - TAIL0
BPT = 488
SB = 1
SLABW = SB * 128
SROWS = 16400
DUMP = 16384
BIG = 0x3FFFFFFF
BPB = 16
NBIN = 31
NBUF = 7
IDXC = 2048

_mesh = plsc.VectorSubcoreMesh(core_axis_name="c", subcore_axis_name="s",
                               num_cores=NC, num_subcores=NS)

_IOTA = lambda: lax.iota(jnp.int32, L)


@functools.partial(
    pl.kernel,
    out_type=(jax.ShapeDtypeStruct((B,), jnp.float32),
              jax.ShapeDtypeStruct((SROWS, 128), jnp.float32),
              jax.ShapeDtypeStruct((SROWS, 128), jnp.float32)),
    mesh=_mesh,
    scratch_types=[
        pltpu.VMEM((IDXC,), jnp.int32),
        pltpu.VMEM((B + L,), jnp.int32),
        pltpu.VMEM((B + L,), jnp.int32),
        pltpu.VMEM((NBIN + L,), jnp.int32),
        pltpu.VMEM((NBUF * D, SLABW), jnp.float32),
        pltpu.VMEM((NBUF * SLABW,), jnp.float32),
        pltpu.VMEM((256, 128), jnp.float32),
        pltpu.VMEM((2, 128), jnp.int32),
        pltpu.VMEM((L,), jnp.int32),
        pltpu.VMEM((512,), jnp.int32),
        pltpu.VMEM((B // NW,), jnp.float32),
        pltpu.VMEM((L,), jnp.float32),
        pltpu.SemaphoreType.DMA,
        pltpu.SemaphoreType.DMA,
        pltpu.SemaphoreType.DMA,
        pltpu.SemaphoreType.REGULAR,
    ],
    compiler_params=pltpu.CompilerParams(needs_layout_passes=False),
)
def _fused_kernel(user, item, uft, ift, ubias, ibias,
                  tail_uf, tail_if, tail_ub, tail_ib, gbias,
                  out, uvals, ivals,
                  idxall, rlist, rbin, binoff,
                  slab, bslab, staging, pchunk,
                  tmpr, blkmap, outv, gbv, ssem, bsem, csem, barsem):
    core = lax.axis_index("c")
    tid = lax.axis_index("s")
    lane0 = _IOTA() == 0

    for j in range(2):
        for k in range(128 // L):
            pchunk[j, pl.ds(k * L, L)] = jnp.full((L,), DUMP, jnp.int32)

    @pl.when(core == 0)
    def _():
        _extract_one(user, uft, ubias, tail_uf, tail_ub, uvals, tid,
                     idxall, rlist, rbin, binoff,
                     slab, bslab, staging, pchunk,
                     tmpr, blkmap, ssem, bsem, csem, lane0)

    @pl.when(core == 1)
    def _():
        _extract_one(item, ift, ibias, tail_if, tail_ib, ivals, tid,
                     idxall, rlist, rbin, binoff,
                     slab, bslab, staging, pchunk,
                     tmpr, blkmap, ssem, bsem, csem, lane0)

    plsc.subcore_barrier()
    pl.semaphore_signal(barsem, 1, core_index=1 - core)
    pl.semaphore_wait(barsem, 1)

    wid = tid * NC + core
    base = wid * (B // NW)
    pltpu.sync_copy(gbias.at[pl.ds(0, 1)], gbv.at[pl.ds(0, 1)])
    gb = gbv[...][0]
    slot0 = _IOTA()
    CH = 64
    NCHK = (B // NW) // CH
    RING = 2

    def dfire(j, par):
        src = pl.ds(base + j * CH, CH)
        pltpu.async_copy(uvals.at[src], staging.at[pl.ds(par * CH, CH)],
                         ssem)
        pltpu.async_copy(ivals.at[src],
                         staging.at[pl.ds(128 + par * CH, CH)], bsem)

    def dwait(par):
        pltpu.make_async_copy(uvals.at[pl.ds(0, CH)],
                              staging.at[pl.ds(par * CH, CH)], ssem).wait()
        pltpu.make_async_copy(ivals.at[pl.ds(0, CH)],
                              staging.at[pl.ds(128 + par * CH, CH)],
                              bsem).wait()

    for p in range(RING - 1):
        dfire(p, p)

    def chunk(j, carry):
        par = lax.rem(j, RING)
        dwait(par)

        @pl.when(j + RING - 1 < NCHK)
        def _():
            dfire(j + RING - 1, lax.rem(j + RING - 1, RING))

        def group(g, c2):
            su = par * CH + g * L + slot0
            si = su + 128
            dcol = jnp.full((L,), D, jnp.int32)
            acc = (plsc.load_gather(staging, [su, dcol])
                   + plsc.load_gather(staging, [si, dcol]) + gb)
            for c in range(D):
                cv = jnp.full((L,), c, jnp.int32)
                acc = acc + (plsc.load_gather(staging, [su, cv])
                             * plsc.load_gather(staging, [si, cv]))
            outv[pl.ds(j * CH + g * L, L)] = acc
            return c2

        return lax.fori_loop(0, CH // L, group, carry)

    lax.fori_loop(0, NCHK, chunk, 0)
    pltpu.sync_copy(outv, out.at[pl.ds(base, B // NW)])


def _extract_one(bidx, tbl, bias, tail_t, tail_b, vals, tid,
                 idxall, rlist, rbin, binoff,
                 slab, bslab, staging, pchunk,
                 tmpr, blkmap, ssem, bsem, csem, lane0):
    b0 = tid * BPT
    last = tid == NS - 1
    nsub = jnp.where(last, NBLK - (NS - 1) * BPT, BPT)
    r_lo = b0 * 128
    r_hi = jnp.where(last, N, r_lo + BPT * 128)

    n_w = 0
    for c in range(B // IDXC):
        pltpu.sync_copy(bidx.at[pl.ds(c * IDXC, IDXC)], idxall)

        def filt(g, off, c=c):
            rv = idxall[pl.ds(g * L, L)]
            m = (rv >= r_lo) & (rv < r_hi)
            cnt = plsc.all_reduce_population_count(m)[0]
            pk = ((rv - r_lo) << 14) | (_IOTA() + (c * IDXC + g * L))
            plsc.store_compressed(rlist.at[pl.ds(off, L)], pk, mask=m)
            return off + cnt

        n_w = lax.fori_loop(0, IDXC // L, filt, n_w)
    rlist[pl.ds(n_w, L)] = jnp.full((L,), BIG, jnp.int32)
    ng = (n_w + L - 1) // L

    def bin_one(b, off2):
        bin_lo = (b * BPB * 128) << 14
        bin_hi = ((b + 1) * BPB * 128) << 14

        def bg(g, o2):
            pk = rlist[pl.ds(g * L, L)]
            m = (pk >= bin_lo) & (pk < bin_hi)
            cnt = plsc.all_reduce_population_count(m)[0]
            plsc.store_compressed(rbin.at[pl.ds(o2, L)], pk, mask=m)
            return o2 + cnt

        off2 = lax.fori_loop(0, ng, bg, off2)
        plsc.store_scatter(binoff, [jnp.full((L,), b + 1, jnp.int32)],
                           jnp.full((L,), off2, jnp.int32), mask=lane0)
        return off2

    plsc.store_scatter(binoff, [jnp.zeros((L,), jnp.int32)],
                       jnp.zeros((L,), jnp.int32), mask=lane0)
    off2 = 0
    for b in range(NBIN):
        off2 = bin_one(b, off2)
    rbin[pl.ds(n_w, L)] = jnp.full((L,), BIG, jnp.int32)

    for k in range(512 // L):
        blkmap[pl.ds(k * L, L)] = jnp.zeros((L,), jnp.int32)

    def mark(g, carry):
        pk = rbin[pl.ds(g * L, L)]
        plsc.store_scatter(blkmap, [(pk >> 21) & 511],
                           jnp.full((L,), 1, jnp.int32))
        return carry

    lax.fori_loop(0, ng, mark, 0)

    def block_live(s):
        return plsc.load_gather(blkmap, [jnp.full((L,), s, jnp.int32)])[0]

    def fire(s, par):
        @pl.when(block_live(s) > 0)
        def _():
            c0 = (b0 + s * SB) * 128
            pltpu.async_copy(tbl.at[:, pl.ds(c0, SLABW)],
                             slab.at[pl.ds(par * D, D)], ssem)
            pltpu.async_copy(bias.at[pl.ds(c0, SLABW)],
                             bslab.at[pl.ds(par * SLABW, SLABW)], bsem)

    def wait(s, par):
        @pl.when(block_live(s) > 0)
        def _():
            pltpu.make_async_copy(tbl.at[:, pl.ds(0, SLABW)],
                                  slab.at[pl.ds(par * D, D)], ssem).wait()
            pltpu.make_async_copy(bias.at[pl.ds(0, SLABW)],
                                  bslab.at[pl.ds(par * SLABW, SLABW)],
                                  bsem).wait()

    def flush(cpar):
        pltpu.async_copy(staging.at[pl.ds(cpar * 128, 128)],
                         vals.at[pchunk.at[cpar]], csem)

    def drain_chunk(cpar):
        pltpu.make_async_copy(staging.at[pl.ds(cpar * 128, 128)],
                              vals.at[pchunk.at[cpar]], csem).wait()

    def do_match(t, carry, sub_rel, spar):
        sc = carry
        slot = lax.rem(sc, 128)
        cpar = lax.rem(sc // 128, 2)

        @pl.when((slot == 0) & (sc >= 128))
        def _():
            drain_chunk(1 - cpar)
            for k in range(128 // L):
                pchunk[cpar, pl.ds(k * L, L)] = jnp.full((L,), DUMP,
                                                         jnp.int32)

        pk_s = plsc.load_gather(tmpr, [jnp.full((L,), t, jnp.int32)])[0]
        p_s = pk_s & 0x3FFF
        off = jnp.full((L,), (pk_s >> 14) - sub_rel, jnp.int32)
        row = cpar * 128 + slot
        for k in range(D // L):
            cv = _IOTA() + (spar * D + k * L)
            staging[row, pl.ds(k * L, L)] = plsc.load_gather(slab, [cv, off])
        bv = plsc.load_gather(bslab, [jnp.full((L,), spar * SLABW, jnp.int32)
                                      + off])
        staging[row, pl.ds(D, L)] = bv
        plsc.store_scatter(pchunk, [jnp.full((L,), cpar, jnp.int32),
                                    jnp.full((L,), slot, jnp.int32)],
                           jnp.full((L,), p_s, jnp.int32), mask=lane0)

        @pl.when(slot == 127)
        def _():
            flush(cpar)

        return sc + 1

    def scan_groups(sub_rel, sub_rel_hi, spar, sc, bst, bend):
        def grp(g, carry):
            pk = rbin[pl.ds(bst + g * L, L)]
            m2 = (pk >= (sub_rel << 14)) & (pk < (sub_rel_hi << 14))
            pc2 = plsc.all_reduce_population_count(m2)[0]

            def has(carry):
                plsc.store_compressed(tmpr.at[pl.ds(0, L)], pk, mask=m2)
                return lax.fori_loop(
                    0, pc2, lambda t, c: do_match(t, c, sub_rel, spar),
                    carry)

            return lax.cond(pc2 > 0, has, lambda c: c, carry)

        ng2 = (bend - bst + L - 1) // L
        return lax.fori_loop(0, ng2, grp, sc)

    def bin_bounds_aligned(b):
        bst, bend = bin_bounds(b)
        return bst & ~(L - 1), bend

    def bin_bounds(b):
        bst = plsc.load_gather(binoff, [jnp.full((L,), b, jnp.int32)])[0]
        bend = plsc.load_gather(binoff,
                                [jnp.full((L,), b + 1, jnp.int32)])[0]
        return bst, bend

    for p in range(NBUF - 1):
        fire(p, p)

    def subslab(s, sc):
        par = lax.rem(s, NBUF)
        wait(s, par)

        @pl.when(s + NBUF - 1 < nsub)
        def _():
            fire(s + NBUF - 1, lax.rem(s + NBUF - 1, NBUF))

        sub_rel = s * SLABW
        bst, bend = bin_bounds_aligned(s // BPB)
        return scan_groups(sub_rel, sub_rel + SLABW, par, sc, bst, bend)

    sc = lax.fori_loop(0, nsub, subslab, 0)

    def tail(sc):
        pltpu.sync_copy(tail_t, slab.at[pl.ds(0, D), pl.ds(0, 128)])
        pltpu.sync_copy(tail_b, bslab.at[pl.ds(0, 128)])
        bst, bend = bin_bounds_aligned(NBIN - 1)
        return scan_groups(TAIL0 - r_lo, N - r_lo, 0, sc, bst, bend)

    sc = lax.cond(tid == NS - 1, tail, lambda c: c, sc)

    @pl.when(lax.rem(sc, 128) > 0)
    def _():
        flush(lax.rem(sc // 128, 2))

    @pl.when(sc > 0)
    def _():
        drain_chunk(lax.rem((sc - 1) // 128, 2))


def kernel(user, item, user_factors, item_factors, user_biases, item_biases,
           global_bias):
    uft = user_factors.T
    ift = item_factors.T
    tail_uf = jnp.pad(user_factors[TAIL0:].T, ((0, 0), (0, 128 - TAILN)))
    tail_if = jnp.pad(item_factors[TAIL0:].T, ((0, 0), (0, 128 - TAILN)))
    tail_ub = jnp.pad(user_biases[TAIL0:, 0], (0, 128 - TAILN))
    tail_ib = jnp.pad(item_biases[TAIL0:, 0], (0, 128 - TAILN))
    out, _, _ = _fused_kernel(user, item, uft, ift,
                              user_biases.reshape(-1),
                              item_biases.reshape(-1),
                              tail_uf, tail_if, tail_ub, tail_ib,
                              global_bias)
    return out
